# Initial kernel scaffold; baseline (speedup 1.0000x reference)
import jax
import jax.numpy as jnp
from jax.experimental import pallas as pl


def kernel(x, edge_attr, edge_index, Wq0, bq0, Wk0, bk0, Wv0, bv0, Ww0, bw0, We0, be0, Wq1, bq1, Wk1, bk1, Wv1, bv1, Ww1, bw1, We1, be1, gamma, beta):
    raise NotImplementedError("write your pallas kernel here")


# speedup vs baseline: 3.5749x; 3.5749x over previous
import functools

import jax
import jax.numpy as jnp
from jax import lax
from jax.experimental import pallas as pl
from jax.experimental.pallas import tpu as pltpu
from jax.experimental.pallas import tpu_sc as plsc

_N = 10000
_E = 320000
_D = 128
_CW = 128
_NCHUNK = _E // _CW
_RB = 2000
_NB = 400
_NPAD = 10240

_f32 = jnp.float32
_bf16 = jnp.bfloat16


def _mesh():
    return plsc.VectorSubcoreMesh(core_axis_name="c", subcore_axis_name="s")


def _ln(v, g, b):
    mu = jnp.mean(v, axis=-1, keepdims=True)
    var = jnp.mean((v - mu) ** 2, axis=-1, keepdims=True)
    return (v - mu) * lax.rsqrt(var + 1e-5) * g + b


def _dot(a16, w16):
    return jnp.dot(a16, w16, preferred_element_type=_f32)


def _sc_gather(table, src2d):
    @functools.partial(
        pl.kernel,
        out_type=jax.ShapeDtypeStruct((_E, _D), _f32),
        mesh=_mesh(),
        scratch_types=[
            pltpu.VMEM((1, _CW), jnp.int32),
            pltpu.VMEM((_CW, _D), _f32),
        ],
    )
    def k(table_hbm, idx_hbm, out_hbm, idx_v, buf_v):
        wid = lax.axis_index("c") * 16 + lax.axis_index("s")
        nk = jnp.where(wid < 4, 79, 78)

        def body(kk, carry):
            j = wid + 32 * kk
            pltpu.sync_copy(idx_hbm.at[j], idx_v)
            pltpu.sync_copy(table_hbm.at[idx_v.at[0]], buf_v)
            pltpu.sync_copy(buf_v, out_hbm.at[pl.ds(j * _CW, _CW)])
            return carry

        lax.fori_loop(0, nk, body, 0)

    return k(table, src2d)


def _sc_attn(KV0, KV1, Q, M, dst2d, zrows):
    out_t = [jax.ShapeDtypeStruct((_N, _D), _f32)] * 2

    @functools.partial(
        pl.kernel,
        out_type=out_t,
        mesh=_mesh(),
        scratch_types=[
            pltpu.VMEM((1, _CW), jnp.int32),
            pltpu.VMEM((_CW, _D), _f32),
            pltpu.VMEM((_CW, _D), _f32),
            pltpu.VMEM((_CW, _D), _f32),
            pltpu.VMEM((1, _D), _f32),
            pltpu.VMEM_SHARED((_N, _D), _f32),
        ],
    )
    def k(kv0h, kv1h, qh, mh, dsth, zh, o0, o1,
          idx_v, kv_v, qd_v, a_v, m_v, acc):
        c = lax.axis_index("c")
        s = lax.axis_index("s")
        pltpu.sync_copy(zh, acc.at[pl.ds(s * 624, 624)])

        @pl.when(s == 15)
        def _():
            pltpu.sync_copy(zh.at[pl.ds(0, 16)], acc.at[pl.ds(9984, 16)])

        plsc.subcore_barrier()
        nk = jnp.where(s < 4, 157, 156)
        pltpu.sync_copy(mh, m_v)

        def run(kvh, coff):
            ms = [m_v[pl.ds(0, 1), pl.ds(coff + 16 * cc, 16)]
                  for cc in range(4)]

            def body(kk, carry):
                j = s + 16 * kk
                pltpu.sync_copy(dsth.at[j], idx_v)
                pltpu.sync_copy(kvh.at[pl.ds(j * _CW, _CW)], kv_v)
                pltpu.sync_copy(qh.at[idx_v.at[0]], qd_v)

                @pl.loop(0, _CW)
                def _(r):
                    for cc in range(4):
                        sl = (pl.ds(r, 1), pl.ds(16 * cc, 16))
                        slq = (pl.ds(r, 1), pl.ds(coff + 16 * cc, 16))
                        slv = (pl.ds(r, 1), pl.ds(64 + 16 * cc, 16))
                        a = jnp.exp(qd_v[slq] * kv_v[sl] - ms[cc])
                        a_v[sl] = a
                        a_v[slv] = a * kv_v[slv]

                pltpu.sync_copy(a_v, acc.at[idx_v.at[0]], add=True)
                return carry

            lax.fori_loop(0, nk, body, 0)

        @pl.when(c == 0)
        def _():
            run(kv0h, 0)

        @pl.when(c == 1)
        def _():
            run(kv1h, 64)

        plsc.subcore_barrier()

        def writeout(oo):
            pltpu.sync_copy(acc.at[pl.ds(s * 624, 624)],
                            oo.at[pl.ds(s * 624, 624)])

            @pl.when(s == 15)
            def _():
                pltpu.sync_copy(acc.at[pl.ds(9984, 16)],
                                oo.at[pl.ds(9984, 16)])

        @pl.when(c == 0)
        def _():
            writeout(o0)

        @pl.when(c == 1)
        def _():
            writeout(o1)

    return k(KV0, KV1, Q, M, dst2d, zrows)


def _sc_edgec(P, R, Te, hpost, src2d, dst2d):
    with_h = hpost is not None
    out_t = [jax.ShapeDtypeStruct((_E, _D), _f32)]
    scratch = [
        pltpu.VMEM((1, _CW), jnp.int32),
        pltpu.VMEM((1, _CW), jnp.int32),
        pltpu.VMEM((_CW, _D), _f32),
        pltpu.VMEM((_CW, _D), _f32),
        pltpu.VMEM((_CW, _D), _f32),
    ]
    if with_h:
        out_t = out_t + [jax.ShapeDtypeStruct((_E, _D), _f32)]
        scratch = scratch + [pltpu.VMEM((_CW, _D), _f32)]

    def body_fn(ph, rh, teh, *rest):
        if with_h:
            hh, srch, dsth, eo, ho, idxs, idxd, tb, pb, rb, hb = rest
        else:
            srch, dsth, eo, idxs, idxd, tb, pb, rb = rest
        wid = lax.axis_index("c") * 16 + lax.axis_index("s")
        nk = jnp.where(wid < 4, 79, 78)

        def body(kk, carry):
            j = wid + 32 * kk
            pltpu.sync_copy(srch.at[j], idxs)
            pltpu.sync_copy(dsth.at[j], idxd)
            pltpu.sync_copy(teh.at[pl.ds(j * _CW, _CW)], tb)
            pltpu.sync_copy(ph.at[idxs.at[0]], pb)
            pltpu.sync_copy(rh.at[idxd.at[0]], rb)
            if with_h:
                pltpu.sync_copy(hh.at[idxs.at[0]], hb)
                pltpu.sync_copy(hb, ho.at[pl.ds(j * _CW, _CW)])

            @pl.loop(0, _CW)
            def _(r):
                for cc in range(8):
                    sl = (pl.ds(r, 1), pl.ds(16 * cc, 16))
                    tb[sl] = tb[sl] + pb[sl] + rb[sl]

            pltpu.sync_copy(tb, eo.at[pl.ds(j * _CW, _CW)])
            return carry

        lax.fori_loop(0, nk, body, 0)

    k = functools.partial(pl.kernel, out_type=out_t, mesh=_mesh(),
                          scratch_types=scratch)(body_fn)
    if with_h:
        return k(P, R, Te, hpost, src2d, dst2d)
    return k(P, R, Te, src2d, dst2d)


def _tc_qpass(h, wq16, bq):
    def body(h_ref, w_ref, b_ref, q_ref, qm_ref):
        i = pl.program_id(0)
        q = _dot(h_ref[...].astype(_bf16), w_ref[...]) + b_ref[...]
        q_ref[...] = q
        bm = jnp.max(jnp.abs(q), axis=0, keepdims=True)
        prev = jnp.where(i == 0, jnp.zeros_like(bm), qm_ref[...])
        qm_ref[...] = jnp.maximum(prev, bm)

    return pl.pallas_call(
        body,
        grid=(_N // _NB,),
        in_specs=[
            pl.BlockSpec((_NB, _D), lambda i: (i, 0)),
            pl.BlockSpec((_D, _D), lambda i: (0, 0)),
            pl.BlockSpec((1, _D), lambda i: (0, 0)),
        ],
        out_specs=[
            pl.BlockSpec((_NB, _D), lambda i: (i, 0)),
            pl.BlockSpec((1, _D), lambda i: (0, 0)),
        ],
        out_shape=[
            jax.ShapeDtypeStruct((_N, _D), _f32),
            jax.ShapeDtypeStruct((1, _D), _f32),
        ],
    )(h, wq16, bq)


def _tc_edge(hs, ein, wk1, wk2, wv1, wv2, wec, bk, bv, be, g, b, second):
    def body(hs_ref, e_ref, wk1r, wk2r, wv1r, wv2r, wecr, bkr, bvr, ber,
             gr, br, kv0r, kv1r, ter, kmr):
        i = pl.program_id(0)
        e_blk = e_ref[...]
        if second:
            e_blk = _ln(jnp.maximum(e_blk, 0.0), gr[...], br[...])
        hsb = hs_ref[...]
        st16 = (hsb * e_blk).astype(_bf16)
        hs16 = hsb.astype(_bf16)
        e16 = e_blk.astype(_bf16)
        kk = _dot(st16, wk1r[...]) + _dot(hs16, wk2r[...]) + bkr[...]
        vv = _dot(st16, wv1r[...]) + _dot(hs16, wv2r[...]) + bvr[...]
        ter[...] = _dot(e16, wecr[...]) + ber[...]
        kv0r[...] = jnp.concatenate([kk[:, :64], vv[:, :64]], axis=1)
        kv1r[...] = jnp.concatenate([kk[:, 64:], vv[:, 64:]], axis=1)
        bm = jnp.max(jnp.abs(kk), axis=0, keepdims=True)
        prev = jnp.where(i == 0, jnp.zeros_like(bm), kmr[...])
        kmr[...] = jnp.maximum(prev, bm)

    full = pl.BlockSpec((_D, _D), lambda i: (0, 0))
    row = pl.BlockSpec((1, _D), lambda i: (0, 0))
    eb = pl.BlockSpec((_RB, _D), lambda i: (i, 0))
    return pl.pallas_call(
        body,
        grid=(_E // _RB,),
        in_specs=[eb, eb, full, full, full, full, full, row, row, row,
                  row, row],
        out_specs=[eb, eb, eb, row],
        out_shape=[
            jax.ShapeDtypeStruct((_E, _D), _f32),
            jax.ShapeDtypeStruct((_E, _D), _f32),
            jax.ShapeDtypeStruct((_E, _D), _f32),
            jax.ShapeDtypeStruct((1, _D), _f32),
        ],
    )(hs, ein, wk1, wk2, wv1, wv2, wec, bk, bv, be, g, b)


def _tc_node(O0, O1, h, ww1, ww2, bw, wea, web, g, b, wqn=None, bqn=None):
    first = wqn is not None

    def body(*refs):
        if first:
            (o0r, o1r, hr, ww1r, ww2r, bwr, wear, webr, gr, br,
             wqr, bqr, hpr, pr, rr, qr, qmr) = refs
        else:
            (o0r, o1r, hr, ww1r, ww2r, bwr, wear, webr, gr, br,
             hpr, pr, rr) = refs
        i = pl.program_id(0)
        o0 = o0r[...]
        o1 = o1r[...]
        ss = jnp.concatenate([o0[:, :64], o1[:, :64]], axis=1)
        tt = jnp.concatenate([o0[:, 64:], o1[:, 64:]], axis=1)
        pos = ss > 0.0
        hn = jnp.where(pos, tt / jnp.where(pos, ss, 1.0), 0.0)
        h16 = hr[...].astype(_bf16)
        h_new = _dot(hn.astype(_bf16), ww1r[...]) + _dot(h16, ww2r[...]) \
            + bwr[...]
        hn16 = h_new.astype(_bf16)
        pr[...] = _dot(hn16, wear[...])
        rr[...] = _dot(hn16, webr[...])
        hp = _ln(jnp.maximum(h_new, 0.0), gr[...], br[...])
        hpr[...] = hp
        if first:
            q = _dot(hp.astype(_bf16), wqr[...]) + bqr[...]
            qr[...] = q
            bm = jnp.max(jnp.abs(q), axis=0, keepdims=True)
            prev = jnp.where(i == 0, jnp.zeros_like(bm), qmr[...])
            qmr[...] = jnp.maximum(prev, bm)

    full = pl.BlockSpec((_D, _D), lambda i: (0, 0))
    row = pl.BlockSpec((1, _D), lambda i: (0, 0))
    nb = pl.BlockSpec((_NB, _D), lambda i: (i, 0))
    in_specs = [nb, nb, nb, full, full, row, full, full, row, row]
    out_specs = [nb, nb, nb]
    out_shape = [jax.ShapeDtypeStruct((_N, _D), _f32)] * 3
    args = [O0, O1, h, ww1, ww2, bw, wea, web, g, b]
    if first:
        in_specs = in_specs + [full, row]
        out_specs = out_specs + [nb, row]
        out_shape = out_shape + [
            jax.ShapeDtypeStruct((_N, _D), _f32),
            jax.ShapeDtypeStruct((1, _D), _f32),
        ]
        args = args + [wqn, bqn]
    return pl.pallas_call(
        body,
        grid=(_N // _NB,),
        in_specs=in_specs,
        out_specs=out_specs,
        out_shape=out_shape,
    )(*args)


def _tc_final(esum, g, b):
    def body(e_ref, gr, br, o_ref):
        o_ref[...] = _ln(jnp.maximum(e_ref[...], 0.0), gr[...], br[...])

    eb = pl.BlockSpec((_RB, _D), lambda i: (i, 0))
    row = pl.BlockSpec((1, _D), lambda i: (0, 0))
    return pl.pallas_call(
        body,
        grid=(_E // _RB,),
        in_specs=[eb, row, row],
        out_specs=eb,
        out_shape=jax.ShapeDtypeStruct((_E, _D), _f32),
    )(esum, g, b)


def kernel(x, edge_attr, edge_index, Wq0, bq0, Wk0, bk0, Wv0, bv0, Ww0,
           bw0, We0, be0, Wq1, bq1, Wk1, bk1, Wv1, bv1, Ww1, bw1, We1,
           be1, gamma, beta):
    ei = edge_index.reshape(2, _NCHUNK, 1, _CW)
    src2d, dst2d = ei[0], ei[1]
    zrows = jnp.zeros((624, _D), _f32)
    g = gamma.reshape(1, _D)
    b = beta.reshape(1, _D)

    def w16(w):
        return w.astype(_bf16)

    q, qmax = _tc_qpass(x, w16(Wq0), bq0.reshape(1, _D))
    hs = _sc_gather(x, src2d)
    kv0, kv1, te, kmax = _tc_edge(
        hs, edge_attr, w16(Wk0[:128]), w16(Wk0[128:]), w16(Wv0[:128]),
        w16(Wv0[128:]), w16(We0[256:]), bk0.reshape(1, _D),
        bv0.reshape(1, _D), be0.reshape(1, _D), g, b, second=False)
    o0, o1 = _sc_attn(kv0, kv1, q, qmax * kmax, dst2d, zrows)
    h1, p1, r1, q, qmax = _tc_node(
        o0, o1, x, w16(Ww0[:128]), w16(Ww0[128:]),
        bw0.reshape(1, _D), w16(We0[:128]), w16(We0[128:256]), g, b,
        w16(Wq1), bq1.reshape(1, _D))
    esum1, hs2 = _sc_edgec(p1, r1, te, h1, src2d, dst2d)

    kv0, kv1, te, kmax = _tc_edge(
        hs2, esum1, w16(Wk1[:128]), w16(Wk1[128:]), w16(Wv1[:128]),
        w16(Wv1[128:]), w16(We1[256:]), bk1.reshape(1, _D),
        bv1.reshape(1, _D), be1.reshape(1, _D), g, b, second=True)
    o0, o1 = _sc_attn(kv0, kv1, q, qmax * kmax, dst2d, zrows)
    h2, p2, r2 = _tc_node(
        o0, o1, h1, w16(Ww1[:128]), w16(Ww1[128:]),
        bw1.reshape(1, _D), w16(We1[:128]), w16(We1[128:256]), g, b)
    (esum2,) = _sc_edgec(p2, r2, te, None, src2d, dst2d)
    e_out = _tc_final(esum2, g, b)
    return h2, e_out


# speedup vs baseline: 5.5395x; 1.5495x over previous
import functools

import jax
import jax.numpy as jnp
from jax import lax
from jax.experimental import pallas as pl
from jax.experimental.pallas import tpu as pltpu
from jax.experimental.pallas import tpu_sc as plsc

_N = 10000
_E = 320000
_D = 128
_CW = 128
_NCHUNK = _E // _CW
_RB = 2000
_NB = 400
_NPAD = 10240

_f32 = jnp.float32
_bf16 = jnp.bfloat16


def _mesh():
    return plsc.VectorSubcoreMesh(core_axis_name="c", subcore_axis_name="s")


def _ln(v, g, b):
    mu = jnp.mean(v, axis=-1, keepdims=True)
    var = jnp.mean((v - mu) ** 2, axis=-1, keepdims=True)
    return (v - mu) * lax.rsqrt(var + 1e-5) * g + b


def _dot(a16, w16):
    return jnp.dot(a16, w16, preferred_element_type=_f32)


def _sc_gather(table, src2d):
    @functools.partial(
        pl.kernel,
        out_type=jax.ShapeDtypeStruct((_E, _D), _f32),
        mesh=_mesh(),
        scratch_types=[
            pltpu.VMEM((79, 1, _CW), jnp.int32),
            pltpu.VMEM((_CW, _D), _f32),
            pltpu.VMEM((_CW, _D), _f32),
            pltpu.SemaphoreType.DMA,
            pltpu.SemaphoreType.DMA,
        ],
    )
    def k(table_hbm, idx_hbm, out_hbm, idx_a, buf0, buf1, sem0, sem1):
        wid = lax.axis_index("c") * 16 + lax.axis_index("s")
        nk = jnp.where(wid < 4, 79, 78)
        j0 = wid * 78 + jnp.minimum(wid, 4)
        pltpu.sync_copy(idx_hbm.at[pl.ds(j0, 78)], idx_a.at[pl.ds(0, 78)])

        @pl.when(wid < 4)
        def _():
            pltpu.sync_copy(idx_hbm.at[j0 + 78], idx_a.at[78])

        pltpu.async_copy(table_hbm.at[idx_a.at[0, 0]], buf0, sem0)

        def body(kk, carry):
            even = kk % 2 == 0

            def step(buf, sem, obuf, osem):
                @pl.when(kk + 1 < nk)
                def _():
                    pltpu.async_copy(
                        table_hbm.at[idx_a.at[kk + 1, 0]], obuf, osem)
                pltpu.make_async_copy(
                    table_hbm.at[idx_a.at[kk, 0]], buf, sem).wait()
                pltpu.sync_copy(buf, out_hbm.at[pl.ds((j0 + kk) * _CW, _CW)])

            @pl.when(even)
            def _():
                step(buf0, sem0, buf1, sem1)

            @pl.when(jnp.logical_not(even))
            def _():
                step(buf1, sem1, buf0, sem0)

            return carry

        lax.fori_loop(0, nk, body, 0)

    return k(table, src2d)


def _sc_attn(KV0, KV1, Q, M, dst2d, zrows):
    out_t = [jax.ShapeDtypeStruct((_N, _D), _f32)] * 2

    @functools.partial(
        pl.kernel,
        out_type=out_t,
        mesh=_mesh(),
        scratch_types=[
            pltpu.VMEM((1, _CW), jnp.int32),
            pltpu.VMEM((1, _CW), jnp.int32),
            pltpu.VMEM((_CW, _D), _f32),
            pltpu.VMEM((_CW, _D), _f32),
            pltpu.VMEM((_CW, _D), _f32),
            pltpu.VMEM((1, _D), _f32),
            pltpu.VMEM_SHARED((_N, _D), _f32),
            pltpu.SemaphoreType.DMA,
            pltpu.SemaphoreType.DMA,
            pltpu.SemaphoreType.DMA,
            pltpu.SemaphoreType.DMA,
            pltpu.SemaphoreType.DMA,
        ],
    )
    def k(kv0h, kv1h, qh, mh, dsth, zh, o0, o1,
          idx0, idx1, kv0v, kv1v, qd_v, m_v, acc,
          si0, si1, sk0, sk1, sq):
        c = lax.axis_index("c")
        s = lax.axis_index("s")
        pltpu.sync_copy(zh, acc.at[pl.ds(s * 624, 624)])

        @pl.when(s == 15)
        def _():
            pltpu.sync_copy(zh.at[pl.ds(0, 16)], acc.at[pl.ds(9984, 16)])

        plsc.subcore_barrier()
        nk = jnp.where(s < 4, 157, 156)
        j0 = s * 156 + jnp.minimum(s, 4)
        pltpu.sync_copy(mh, m_v)

        def jrow(kk):
            return jnp.minimum(j0 + kk, _NCHUNK - 1)

        def run(kvh, coff):
            ms = [m_v[pl.ds(0, 1), pl.ds(coff + 16 * cc, 16)]
                  for cc in range(4)]

            def issue_idx(kk, ib, si):
                pltpu.async_copy(dsth.at[jrow(kk)], ib, si)

            def issue_kv(kk, kvb, sk):
                pltpu.async_copy(
                    kvh.at[pl.ds(jrow(kk) * _CW, _CW)], kvb, sk)

            issue_idx(0, idx0, si0)
            pltpu.make_async_copy(dsth.at[jrow(0)], idx0, si0).wait()
            issue_kv(0, kv0v, sk0)
            pltpu.async_copy(qh.at[idx0.at[0]], qd_v, sq)
            issue_idx(1, idx1, si1)

            def step(kk, ib, si, oib, osi, kvb, sk, okvb, osk):
                @pl.when(kk + 1 < nk)
                def _():
                    issue_kv(kk + 1, okvb, osk)

                pltpu.make_async_copy(
                    kvh.at[pl.ds(jrow(kk) * _CW, _CW)], kvb, sk).wait()
                pltpu.make_async_copy(qh.at[ib.at[0]], qd_v, sq).wait()

                @pl.loop(0, _CW)
                def _(r):
                    for cc in range(4):
                        sl = (pl.ds(r, 1), pl.ds(16 * cc, 16))
                        slq = (pl.ds(r, 1), pl.ds(coff + 16 * cc, 16))
                        slv = (pl.ds(r, 1), pl.ds(64 + 16 * cc, 16))
                        a = jnp.exp(qd_v[slq] * kvb[sl] - ms[cc])
                        kvb[sl] = a
                        kvb[slv] = a * kvb[slv]

                @pl.when(kk + 1 < nk)
                def _():
                    pltpu.make_async_copy(dsth.at[jrow(kk + 1)], oib,
                                          osi).wait()
                    pltpu.async_copy(qh.at[oib.at[0]], qd_v, sq)

                pltpu.sync_copy(kvb, acc.at[ib.at[0]], add=True)

                @pl.when(kk + 2 < nk)
                def _():
                    issue_idx(kk + 2, ib, si)

            def body(kk, carry):
                @pl.when(kk % 2 == 0)
                def _():
                    step(kk, idx0, si0, idx1, si1, kv0v, sk0, kv1v, sk1)

                @pl.when(kk % 2 == 1)
                def _():
                    step(kk, idx1, si1, idx0, si0, kv1v, sk1, kv0v, sk0)

                return carry

            lax.fori_loop(0, nk, body, 0)

        @pl.when(c == 0)
        def _():
            run(kv0h, 0)

        @pl.when(c == 1)
        def _():
            run(kv1h, 64)

        plsc.subcore_barrier()

        def writeout(oo):
            pltpu.sync_copy(acc.at[pl.ds(s * 624, 624)],
                            oo.at[pl.ds(s * 624, 624)])

            @pl.when(s == 15)
            def _():
                pltpu.sync_copy(acc.at[pl.ds(9984, 16)],
                                oo.at[pl.ds(9984, 16)])

        @pl.when(c == 0)
        def _():
            writeout(o0)

        @pl.when(c == 1)
        def _():
            writeout(o1)

    return k(KV0, KV1, Q, M, dst2d, zrows)


def _sc_edgec(P, R, hpost, src2d, dst2d):
    with_h = hpost is not None
    out_t = [jax.ShapeDtypeStruct((_E, _D), _f32)]
    scratch = [
        pltpu.VMEM((79, 1, _CW), jnp.int32),
        pltpu.VMEM((79, 1, _CW), jnp.int32),
        pltpu.VMEM((_CW, _D), _f32),
        pltpu.VMEM((_CW, _D), _f32),
        pltpu.VMEM((_CW, _D), _f32),
        pltpu.VMEM((_CW, _D), _f32),
        pltpu.SemaphoreType.DMA,
        pltpu.SemaphoreType.DMA,
        pltpu.SemaphoreType.DMA,
        pltpu.SemaphoreType.DMA,
    ]
    if with_h:
        out_t = out_t + [jax.ShapeDtypeStruct((_E, _D), _f32)]
        scratch = scratch + [pltpu.VMEM((_CW, _D), _f32),
                             pltpu.VMEM((_CW, _D), _f32),
                             pltpu.SemaphoreType.DMA,
                             pltpu.SemaphoreType.DMA]

    def body_fn(ph, rh, *rest):
        if with_h:
            (hh, srch, dsth, eo, ho, idxs_a, idxd_a, pb0, pb1, rb0, rb1,
             sp0, sp1, sr0, sr1, hb0, hb1, sh0, sh1) = rest
        else:
            (srch, dsth, eo, idxs_a, idxd_a, pb0, pb1, rb0, rb1,
             sp0, sp1, sr0, sr1) = rest
        wid = lax.axis_index("c") * 16 + lax.axis_index("s")
        nk = jnp.where(wid < 4, 79, 78)
        j0 = wid * 78 + jnp.minimum(wid, 4)
        pltpu.sync_copy(srch.at[pl.ds(j0, 78)], idxs_a.at[pl.ds(0, 78)])
        pltpu.sync_copy(dsth.at[pl.ds(j0, 78)], idxd_a.at[pl.ds(0, 78)])

        @pl.when(wid < 4)
        def _():
            pltpu.sync_copy(srch.at[j0 + 78], idxs_a.at[78])
            pltpu.sync_copy(dsth.at[j0 + 78], idxd_a.at[78])

        def loads(kk, pb, rb, hb, sp, sr, sh):
            pltpu.async_copy(ph.at[idxs_a.at[kk, 0]], pb, sp)
            pltpu.async_copy(rh.at[idxd_a.at[kk, 0]], rb, sr)
            if with_h:
                pltpu.async_copy(hh.at[idxs_a.at[kk, 0]], hb, sh)

        def step(kk, pb, rb, hb, sp, sr, sh, opb, orb, ohb, osp, osr, osh):
            @pl.when(kk + 1 < nk)
            def _():
                loads(kk + 1, opb, orb, ohb, osp, osr, osh)

            pltpu.make_async_copy(ph.at[idxs_a.at[kk, 0]], pb, sp).wait()
            pltpu.make_async_copy(rh.at[idxd_a.at[kk, 0]], rb, sr).wait()
            if with_h:
                pltpu.make_async_copy(hh.at[idxs_a.at[kk, 0]], hb,
                                      sh).wait()
                pltpu.sync_copy(hb, ho.at[pl.ds((j0 + kk) * _CW, _CW)])

            @pl.loop(0, _CW)
            def _(r):
                for cc in range(8):
                    sl = (pl.ds(r, 1), pl.ds(16 * cc, 16))
                    pb[sl] = pb[sl] + rb[sl]

            pltpu.sync_copy(pb, eo.at[pl.ds((j0 + kk) * _CW, _CW)])

        if with_h:
            loads(0, pb0, rb0, hb0, sp0, sr0, sh0)
        else:
            loads(0, pb0, rb0, None, sp0, sr0, None)

        def body(kk, carry):
            if with_h:
                @pl.when(kk % 2 == 0)
                def _():
                    step(kk, pb0, rb0, hb0, sp0, sr0, sh0,
                         pb1, rb1, hb1, sp1, sr1, sh1)

                @pl.when(kk % 2 == 1)
                def _():
                    step(kk, pb1, rb1, hb1, sp1, sr1, sh1,
                         pb0, rb0, hb0, sp0, sr0, sh0)
            else:
                @pl.when(kk % 2 == 0)
                def _():
                    step(kk, pb0, rb0, None, sp0, sr0, None,
                         pb1, rb1, None, sp1, sr1, None)

                @pl.when(kk % 2 == 1)
                def _():
                    step(kk, pb1, rb1, None, sp1, sr1, None,
                         pb0, rb0, None, sp0, sr0, None)
            return carry

        lax.fori_loop(0, nk, body, 0)

    k = functools.partial(pl.kernel, out_type=out_t, mesh=_mesh(),
                          scratch_types=scratch)(body_fn)
    if with_h:
        return k(P, R, hpost, src2d, dst2d)
    return k(P, R, src2d, dst2d)


def _tc_qpass(h, wq16, bq):
    def body(h_ref, w_ref, b_ref, q_ref, qm_ref):
        i = pl.program_id(0)
        q = _dot(h_ref[...].astype(_bf16), w_ref[...]) + b_ref[...]
        q_ref[...] = q
        bm = jnp.max(jnp.abs(q), axis=0, keepdims=True)
        prev = jnp.where(i == 0, jnp.zeros_like(bm), qm_ref[...])
        qm_ref[...] = jnp.maximum(prev, bm)

    return pl.pallas_call(
        body,
        grid=(_N // _NB,),
        in_specs=[
            pl.BlockSpec((_NB, _D), lambda i: (i, 0)),
            pl.BlockSpec((_D, _D), lambda i: (0, 0)),
            pl.BlockSpec((1, _D), lambda i: (0, 0)),
        ],
        out_specs=[
            pl.BlockSpec((_NB, _D), lambda i: (i, 0)),
            pl.BlockSpec((1, _D), lambda i: (0, 0)),
        ],
        out_shape=[
            jax.ShapeDtypeStruct((_N, _D), _f32),
            jax.ShapeDtypeStruct((1, _D), _f32),
        ],
    )(h, wq16, bq)


def _tc_edge(hs, ein, wk1, wk2, wv1, wv2, wec, bk, bv, be, g, b, second,
             tprev=None):
    def body(hs_ref, e_ref, *refs):
        if second:
            (tpr, wk1r, wk2r, wv1r, wv2r, wecr, bkr, bvr, ber,
             gr, br, kv0r, kv1r, ter, kmr) = refs
        else:
            (wk1r, wk2r, wv1r, wv2r, wecr, bkr, bvr, ber,
             gr, br, kv0r, kv1r, ter, kmr) = refs
        i = pl.program_id(0)
        e_blk = e_ref[...]
        if second:
            e_blk = _ln(jnp.maximum(e_blk + tpr[...], 0.0), gr[...],
                        br[...])
        hsb = hs_ref[...]
        st16 = (hsb * e_blk).astype(_bf16)
        hs16 = hsb.astype(_bf16)
        e16 = e_blk.astype(_bf16)
        kk = _dot(st16, wk1r[...]) + _dot(hs16, wk2r[...]) + bkr[...]
        vv = _dot(st16, wv1r[...]) + _dot(hs16, wv2r[...]) + bvr[...]
        ter[...] = _dot(e16, wecr[...]) + ber[...]
        kv0r[...] = jnp.concatenate([kk[:, :64], vv[:, :64]], axis=1)
        kv1r[...] = jnp.concatenate([kk[:, 64:], vv[:, 64:]], axis=1)
        bm = jnp.max(jnp.abs(kk), axis=0, keepdims=True)
        prev = jnp.where(i == 0, jnp.zeros_like(bm), kmr[...])
        kmr[...] = jnp.maximum(prev, bm)

    full = pl.BlockSpec((_D, _D), lambda i: (0, 0))
    row = pl.BlockSpec((1, _D), lambda i: (0, 0))
    eb = pl.BlockSpec((_RB, _D), lambda i: (i, 0))
    in_specs = [eb, eb]
    args = [hs, ein]
    if second:
        in_specs = in_specs + [eb]
        args = args + [tprev]
    in_specs = in_specs + [full, full, full, full, full, row, row, row,
                           row, row]
    args = args + [wk1, wk2, wv1, wv2, wec, bk, bv, be, g, b]
    return pl.pallas_call(
        body,
        grid=(_E // _RB,),
        in_specs=in_specs,
        out_specs=[eb, eb, eb, row],
        out_shape=[
            jax.ShapeDtypeStruct((_E, _D), _f32),
            jax.ShapeDtypeStruct((_E, _D), _f32),
            jax.ShapeDtypeStruct((_E, _D), _f32),
            jax.ShapeDtypeStruct((1, _D), _f32),
        ],
    )(*args)


def _tc_node(O0, O1, h, ww1, ww2, bw, wea, web, g, b, wqn=None, bqn=None):
    first = wqn is not None

    def body(*refs):
        if first:
            (o0r, o1r, hr, ww1r, ww2r, bwr, wear, webr, gr, br,
             wqr, bqr, hpr, pr, rr, qr, qmr) = refs
        else:
            (o0r, o1r, hr, ww1r, ww2r, bwr, wear, webr, gr, br,
             hpr, pr, rr) = refs
        i = pl.program_id(0)
        o0 = o0r[...]
        o1 = o1r[...]
        ss = jnp.concatenate([o0[:, :64], o1[:, :64]], axis=1)
        tt = jnp.concatenate([o0[:, 64:], o1[:, 64:]], axis=1)
        pos = ss > 0.0
        hn = jnp.where(pos, tt / jnp.where(pos, ss, 1.0), 0.0)
        h16 = hr[...].astype(_bf16)
        h_new = _dot(hn.astype(_bf16), ww1r[...]) + _dot(h16, ww2r[...]) \
            + bwr[...]
        hn16 = h_new.astype(_bf16)
        pr[...] = _dot(hn16, wear[...])
        rr[...] = _dot(hn16, webr[...])
        hp = _ln(jnp.maximum(h_new, 0.0), gr[...], br[...])
        hpr[...] = hp
        if first:
            q = _dot(hp.astype(_bf16), wqr[...]) + bqr[...]
            qr[...] = q
            bm = jnp.max(jnp.abs(q), axis=0, keepdims=True)
            prev = jnp.where(i == 0, jnp.zeros_like(bm), qmr[...])
            qmr[...] = jnp.maximum(prev, bm)

    full = pl.BlockSpec((_D, _D), lambda i: (0, 0))
    row = pl.BlockSpec((1, _D), lambda i: (0, 0))
    nb = pl.BlockSpec((_NB, _D), lambda i: (i, 0))
    in_specs = [nb, nb, nb, full, full, row, full, full, row, row]
    out_specs = [nb, nb, nb]
    out_shape = [jax.ShapeDtypeStruct((_N, _D), _f32)] * 3
    args = [O0, O1, h, ww1, ww2, bw, wea, web, g, b]
    if first:
        in_specs = in_specs + [full, row]
        out_specs = out_specs + [nb, row]
        out_shape = out_shape + [
            jax.ShapeDtypeStruct((_N, _D), _f32),
            jax.ShapeDtypeStruct((1, _D), _f32),
        ]
        args = args + [wqn, bqn]
    return pl.pallas_call(
        body,
        grid=(_N // _NB,),
        in_specs=in_specs,
        out_specs=out_specs,
        out_shape=out_shape,
    )(*args)


def _tc_final(esum, te, g, b):
    def body(e_ref, t_ref, gr, br, o_ref):
        o_ref[...] = _ln(jnp.maximum(e_ref[...] + t_ref[...], 0.0),
                         gr[...], br[...])

    eb = pl.BlockSpec((_RB, _D), lambda i: (i, 0))
    row = pl.BlockSpec((1, _D), lambda i: (0, 0))
    return pl.pallas_call(
        body,
        grid=(_E // _RB,),
        in_specs=[eb, eb, row, row],
        out_specs=eb,
        out_shape=jax.ShapeDtypeStruct((_E, _D), _f32),
    )(esum, te, g, b)


def kernel(x, edge_attr, edge_index, Wq0, bq0, Wk0, bk0, Wv0, bv0, Ww0,
           bw0, We0, be0, Wq1, bq1, Wk1, bk1, Wv1, bv1, Ww1, bw1, We1,
           be1, gamma, beta):
    ei = edge_index.reshape(2, _NCHUNK, 1, _CW)
    src2d, dst2d = ei[0], ei[1]
    zrows = jnp.zeros((624, _D), _f32)
    g = gamma.reshape(1, _D)
    b = beta.reshape(1, _D)

    def w16(w):
        return w.astype(_bf16)

    q, qmax = _tc_qpass(x, w16(Wq0), bq0.reshape(1, _D))
    hs = _sc_gather(x, src2d)
    kv0, kv1, te, kmax = _tc_edge(
        hs, edge_attr, w16(Wk0[:128]), w16(Wk0[128:]), w16(Wv0[:128]),
        w16(Wv0[128:]), w16(We0[256:]), bk0.reshape(1, _D),
        bv0.reshape(1, _D), be0.reshape(1, _D), g, b, second=False)
    o0, o1 = _sc_attn(kv0, kv1, q, qmax * kmax, dst2d, zrows)
    h1, p1, r1, q, qmax = _tc_node(
        o0, o1, x, w16(Ww0[:128]), w16(Ww0[128:]),
        bw0.reshape(1, _D), w16(We0[:128]), w16(We0[128:256]), g, b,
        w16(Wq1), bq1.reshape(1, _D))
    esum1, hs2 = _sc_edgec(p1, r1, h1, src2d, dst2d)

    kv0, kv1, te, kmax = _tc_edge(
        hs2, esum1, w16(Wk1[:128]), w16(Wk1[128:]), w16(Wv1[:128]),
        w16(Wv1[128:]), w16(We1[256:]), bk1.reshape(1, _D),
        bv1.reshape(1, _D), be1.reshape(1, _D), g, b, second=True,
        tprev=te)
    o0, o1 = _sc_attn(kv0, kv1, q, qmax * kmax, dst2d, zrows)
    h2, p2, r2 = _tc_node(
        o0, o1, h1, w16(Ww1[:128]), w16(Ww1[128:]),
        bw1.reshape(1, _D), w16(We1[:128]), w16(We1[128:256]), g, b)
    (esum2,) = _sc_edgec(p2, r2, None, src2d, dst2d)
    e_out = _tc_final(esum2, te, g, b)
    return h2, e_out


# speedup vs baseline: 6.0302x; 1.0886x over previous
import functools

import jax
import jax.numpy as jnp
from jax import lax
from jax.experimental import pallas as pl
from jax.experimental.pallas import tpu as pltpu
from jax.experimental.pallas import tpu_sc as plsc

_N = 10000
_E = 320000
_D = 128
_CW = 128
_NCHUNK = _E // _CW
_RB = 2000
_NB = 400
_NPAD = 10240

_f32 = jnp.float32
_bf16 = jnp.bfloat16


def _mesh():
    return plsc.VectorSubcoreMesh(core_axis_name="c", subcore_axis_name="s")


def _ln(v, g, b):
    mu = jnp.mean(v, axis=-1, keepdims=True)
    var = jnp.mean((v - mu) ** 2, axis=-1, keepdims=True)
    return (v - mu) * lax.rsqrt(var + 1e-5) * g + b


def _dot(a16, w16):
    return jnp.dot(a16, w16, preferred_element_type=_f32)


def _sc_gather(table, src2d):
    @functools.partial(
        pl.kernel,
        out_type=jax.ShapeDtypeStruct((_E, _D), _f32),
        mesh=_mesh(),
        scratch_types=[
            pltpu.VMEM((79, 1, _CW), jnp.int32),
            pltpu.VMEM((_CW, _D), _f32),
            pltpu.VMEM((_CW, _D), _f32),
            pltpu.SemaphoreType.DMA,
            pltpu.SemaphoreType.DMA,
        ],
    )
    def k(table_hbm, idx_hbm, out_hbm, idx_a, buf0, buf1, sem0, sem1):
        wid = lax.axis_index("c") * 16 + lax.axis_index("s")
        nk = jnp.where(wid < 4, 79, 78)
        j0 = wid * 78 + jnp.minimum(wid, 4)
        pltpu.sync_copy(idx_hbm.at[pl.ds(j0, 78)], idx_a.at[pl.ds(0, 78)])

        @pl.when(wid < 4)
        def _():
            pltpu.sync_copy(idx_hbm.at[j0 + 78], idx_a.at[78])

        pltpu.async_copy(table_hbm.at[idx_a.at[0, 0]], buf0, sem0)

        def body(kk, carry):
            even = kk % 2 == 0

            def step(buf, sem, obuf, osem):
                @pl.when(kk + 1 < nk)
                def _():
                    pltpu.async_copy(
                        table_hbm.at[idx_a.at[kk + 1, 0]], obuf, osem)
                pltpu.make_async_copy(
                    table_hbm.at[idx_a.at[kk, 0]], buf, sem).wait()
                pltpu.sync_copy(buf, out_hbm.at[pl.ds((j0 + kk) * _CW, _CW)])

            @pl.when(even)
            def _():
                step(buf0, sem0, buf1, sem1)

            @pl.when(jnp.logical_not(even))
            def _():
                step(buf1, sem1, buf0, sem0)

            return carry

        lax.fori_loop(0, nk, body, 0)

    return k(table, src2d)


def _sc_attn(KV0, KV1, Q, M, dst2d, zrows):
    out_t = [jax.ShapeDtypeStruct((_N, _D), _f32)] * 2

    @functools.partial(
        pl.kernel,
        out_type=out_t,
        mesh=_mesh(),
        scratch_types=[
            pltpu.VMEM((1, _CW), jnp.int32),
            pltpu.VMEM((1, _CW), jnp.int32),
            pltpu.VMEM((_CW, _D), _f32),
            pltpu.VMEM((_CW, _D), _f32),
            pltpu.VMEM((_CW, _D), _f32),
            pltpu.VMEM((1, _D), _f32),
            pltpu.VMEM_SHARED((_N, _D), _f32),
            pltpu.SemaphoreType.DMA,
            pltpu.SemaphoreType.DMA,
            pltpu.SemaphoreType.DMA,
            pltpu.SemaphoreType.DMA,
            pltpu.SemaphoreType.DMA,
        ],
    )
    def k(kv0h, kv1h, qh, mh, dsth, zh, o0, o1,
          idx0, idx1, kv0v, kv1v, qd_v, m_v, acc,
          si0, si1, sk0, sk1, sq):
        c = lax.axis_index("c")
        s = lax.axis_index("s")
        pltpu.sync_copy(zh, acc.at[pl.ds(s * 624, 624)])

        @pl.when(s == 15)
        def _():
            pltpu.sync_copy(zh.at[pl.ds(0, 16)], acc.at[pl.ds(9984, 16)])

        plsc.subcore_barrier()
        nk = jnp.where(s < 4, 157, 156)
        j0 = s * 156 + jnp.minimum(s, 4)
        pltpu.sync_copy(mh, m_v)

        def jrow(kk):
            return jnp.minimum(j0 + kk, _NCHUNK - 1)

        def run(kvh, coff):
            ms = [m_v[pl.ds(0, 1), pl.ds(coff + 16 * cc, 16)]
                  for cc in range(4)]

            def issue_idx(kk, ib, si):
                pltpu.async_copy(dsth.at[jrow(kk)], ib, si)

            def issue_kv(kk, kvb, sk):
                pltpu.async_copy(
                    kvh.at[pl.ds(jrow(kk) * _CW, _CW)], kvb, sk)

            issue_idx(0, idx0, si0)
            pltpu.make_async_copy(dsth.at[jrow(0)], idx0, si0).wait()
            issue_kv(0, kv0v, sk0)
            pltpu.async_copy(qh.at[idx0.at[0]], qd_v, sq)
            issue_idx(1, idx1, si1)

            def step(kk, ib, si, oib, osi, kvb, sk, okvb, osk):
                @pl.when(kk + 1 < nk)
                def _():
                    issue_kv(kk + 1, okvb, osk)

                pltpu.make_async_copy(
                    kvh.at[pl.ds(jrow(kk) * _CW, _CW)], kvb, sk).wait()
                pltpu.make_async_copy(qh.at[ib.at[0]], qd_v, sq).wait()

                @pl.loop(0, _CW, step=4)
                def _(r0):
                    for dr in range(4):
                        r = r0 + dr
                        for cc in range(4):
                            sl = (pl.ds(r, 1), pl.ds(16 * cc, 16))
                            slq = (pl.ds(r, 1),
                                   pl.ds(coff + 16 * cc, 16))
                            slv = (pl.ds(r, 1),
                                   pl.ds(64 + 16 * cc, 16))
                            a = jnp.exp(qd_v[slq] * kvb[sl] - ms[cc])
                            kvb[sl] = a
                            kvb[slv] = a * kvb[slv]

                @pl.when(kk + 1 < nk)
                def _():
                    pltpu.make_async_copy(dsth.at[jrow(kk + 1)], oib,
                                          osi).wait()
                    pltpu.async_copy(qh.at[oib.at[0]], qd_v, sq)

                pltpu.sync_copy(kvb, acc.at[ib.at[0]], add=True)

                @pl.when(kk + 2 < nk)
                def _():
                    issue_idx(kk + 2, ib, si)

            def body(kk, carry):
                @pl.when(kk % 2 == 0)
                def _():
                    step(kk, idx0, si0, idx1, si1, kv0v, sk0, kv1v, sk1)

                @pl.when(kk % 2 == 1)
                def _():
                    step(kk, idx1, si1, idx0, si0, kv1v, sk1, kv0v, sk0)

                return carry

            lax.fori_loop(0, nk, body, 0)

        @pl.when(c == 0)
        def _():
            run(kv0h, 0)

        @pl.when(c == 1)
        def _():
            run(kv1h, 64)

        plsc.subcore_barrier()

        def writeout(oo):
            pltpu.sync_copy(acc.at[pl.ds(s * 624, 624)],
                            oo.at[pl.ds(s * 624, 624)])

            @pl.when(s == 15)
            def _():
                pltpu.sync_copy(acc.at[pl.ds(9984, 16)],
                                oo.at[pl.ds(9984, 16)])

        @pl.when(c == 0)
        def _():
            writeout(o0)

        @pl.when(c == 1)
        def _():
            writeout(o1)

    return k(KV0, KV1, Q, M, dst2d, zrows)


def _sc_edgec(P, R, hpost, src2d, dst2d):
    with_h = hpost is not None
    out_t = [jax.ShapeDtypeStruct((_E, _D), _f32)]
    scratch = [
        pltpu.VMEM((79, 1, _CW), jnp.int32),
        pltpu.VMEM((79, 1, _CW), jnp.int32),
        pltpu.VMEM((_CW, _D), _f32),
        pltpu.VMEM((_CW, _D), _f32),
        pltpu.VMEM((_CW, _D), _f32),
        pltpu.VMEM((_CW, _D), _f32),
        pltpu.SemaphoreType.DMA,
        pltpu.SemaphoreType.DMA,
        pltpu.SemaphoreType.DMA,
        pltpu.SemaphoreType.DMA,
    ]
    if with_h:
        out_t = out_t + [jax.ShapeDtypeStruct((_E, _D), _f32)]
        scratch = scratch + [pltpu.VMEM((_CW, _D), _f32),
                             pltpu.VMEM((_CW, _D), _f32),
                             pltpu.SemaphoreType.DMA,
                             pltpu.SemaphoreType.DMA]

    def body_fn(ph, rh, *rest):
        if with_h:
            (hh, srch, dsth, eo, ho, idxs_a, idxd_a, pb0, pb1, rb0, rb1,
             sp0, sp1, sr0, sr1, hb0, hb1, sh0, sh1) = rest
        else:
            (srch, dsth, eo, idxs_a, idxd_a, pb0, pb1, rb0, rb1,
             sp0, sp1, sr0, sr1) = rest
        wid = lax.axis_index("c") * 16 + lax.axis_index("s")
        nk = jnp.where(wid < 4, 79, 78)
        j0 = wid * 78 + jnp.minimum(wid, 4)
        pltpu.sync_copy(srch.at[pl.ds(j0, 78)], idxs_a.at[pl.ds(0, 78)])
        pltpu.sync_copy(dsth.at[pl.ds(j0, 78)], idxd_a.at[pl.ds(0, 78)])

        @pl.when(wid < 4)
        def _():
            pltpu.sync_copy(srch.at[j0 + 78], idxs_a.at[78])
            pltpu.sync_copy(dsth.at[j0 + 78], idxd_a.at[78])

        def loads(kk, pb, rb, hb, sp, sr, sh):
            pltpu.async_copy(ph.at[idxs_a.at[kk, 0]], pb, sp)
            pltpu.async_copy(rh.at[idxd_a.at[kk, 0]], rb, sr)
            if with_h:
                pltpu.async_copy(hh.at[idxs_a.at[kk, 0]], hb, sh)

        def step(kk, pb, rb, hb, sp, sr, sh, opb, orb, ohb, osp, osr, osh):
            @pl.when(kk + 1 < nk)
            def _():
                loads(kk + 1, opb, orb, ohb, osp, osr, osh)

            pltpu.make_async_copy(ph.at[idxs_a.at[kk, 0]], pb, sp).wait()
            pltpu.make_async_copy(rh.at[idxd_a.at[kk, 0]], rb, sr).wait()
            if with_h:
                pltpu.make_async_copy(hh.at[idxs_a.at[kk, 0]], hb,
                                      sh).wait()
                pltpu.sync_copy(hb, ho.at[pl.ds((j0 + kk) * _CW, _CW)])

            @pl.loop(0, _CW, step=4)
            def _(r0):
                for dr in range(4):
                    for cc in range(8):
                        sl = (pl.ds(r0 + dr, 1), pl.ds(16 * cc, 16))
                        pb[sl] = pb[sl] + rb[sl]

            pltpu.sync_copy(pb, eo.at[pl.ds((j0 + kk) * _CW, _CW)])

        if with_h:
            loads(0, pb0, rb0, hb0, sp0, sr0, sh0)
        else:
            loads(0, pb0, rb0, None, sp0, sr0, None)

        def body(kk, carry):
            if with_h:
                @pl.when(kk % 2 == 0)
                def _():
                    step(kk, pb0, rb0, hb0, sp0, sr0, sh0,
                         pb1, rb1, hb1, sp1, sr1, sh1)

                @pl.when(kk % 2 == 1)
                def _():
                    step(kk, pb1, rb1, hb1, sp1, sr1, sh1,
                         pb0, rb0, hb0, sp0, sr0, sh0)
            else:
                @pl.when(kk % 2 == 0)
                def _():
                    step(kk, pb0, rb0, None, sp0, sr0, None,
                         pb1, rb1, None, sp1, sr1, None)

                @pl.when(kk % 2 == 1)
                def _():
                    step(kk, pb1, rb1, None, sp1, sr1, None,
                         pb0, rb0, None, sp0, sr0, None)
            return carry

        lax.fori_loop(0, nk, body, 0)

    k = functools.partial(pl.kernel, out_type=out_t, mesh=_mesh(),
                          scratch_types=scratch)(body_fn)
    if with_h:
        return k(P, R, hpost, src2d, dst2d)
    return k(P, R, src2d, dst2d)


def _tc_qpass(h, wq16, bq):
    def body(h_ref, w_ref, b_ref, q_ref, qm_ref):
        i = pl.program_id(0)
        q = _dot(h_ref[...].astype(_bf16), w_ref[...]) + b_ref[...]
        q_ref[...] = q
        bm = jnp.max(jnp.abs(q), axis=0, keepdims=True)
        prev = jnp.where(i == 0, jnp.zeros_like(bm), qm_ref[...])
        qm_ref[...] = jnp.maximum(prev, bm)

    return pl.pallas_call(
        body,
        grid=(_N // _NB,),
        in_specs=[
            pl.BlockSpec((_NB, _D), lambda i: (i, 0)),
            pl.BlockSpec((_D, _D), lambda i: (0, 0)),
            pl.BlockSpec((1, _D), lambda i: (0, 0)),
        ],
        out_specs=[
            pl.BlockSpec((_NB, _D), lambda i: (i, 0)),
            pl.BlockSpec((1, _D), lambda i: (0, 0)),
        ],
        out_shape=[
            jax.ShapeDtypeStruct((_N, _D), _f32),
            jax.ShapeDtypeStruct((1, _D), _f32),
        ],
    )(h, wq16, bq)


def _tc_edge(hs, ein, wk1, wk2, wv1, wv2, wec, bk, bv, be, g, b, second,
             tprev=None):
    def body(hs_ref, e_ref, *refs):
        if second:
            (tpr, wk1r, wk2r, wv1r, wv2r, wecr, bkr, bvr, ber,
             gr, br, kv0r, kv1r, ter, kmr) = refs
        else:
            (wk1r, wk2r, wv1r, wv2r, wecr, bkr, bvr, ber,
             gr, br, kv0r, kv1r, ter, kmr) = refs
        i = pl.program_id(0)
        e_blk = e_ref[...]
        if second:
            e_blk = _ln(jnp.maximum(e_blk + tpr[...], 0.0), gr[...],
                        br[...])
        hsb = hs_ref[...]
        st16 = (hsb * e_blk).astype(_bf16)
        hs16 = hsb.astype(_bf16)
        e16 = e_blk.astype(_bf16)
        kk = _dot(st16, wk1r[...]) + _dot(hs16, wk2r[...]) + bkr[...]
        vv = _dot(st16, wv1r[...]) + _dot(hs16, wv2r[...]) + bvr[...]
        ter[...] = _dot(e16, wecr[...]) + ber[...]
        kv0r[...] = jnp.concatenate([kk[:, :64], vv[:, :64]], axis=1)
        kv1r[...] = jnp.concatenate([kk[:, 64:], vv[:, 64:]], axis=1)
        bm = jnp.max(jnp.abs(kk), axis=0, keepdims=True)
        prev = jnp.where(i == 0, jnp.zeros_like(bm), kmr[...])
        kmr[...] = jnp.maximum(prev, bm)

    full = pl.BlockSpec((_D, _D), lambda i: (0, 0))
    row = pl.BlockSpec((1, _D), lambda i: (0, 0))
    eb = pl.BlockSpec((_RB, _D), lambda i: (i, 0))
    in_specs = [eb, eb]
    args = [hs, ein]
    if second:
        in_specs = in_specs + [eb]
        args = args + [tprev]
    in_specs = in_specs + [full, full, full, full, full, row, row, row,
                           row, row]
    args = args + [wk1, wk2, wv1, wv2, wec, bk, bv, be, g, b]
    return pl.pallas_call(
        body,
        grid=(_E // _RB,),
        in_specs=in_specs,
        out_specs=[eb, eb, eb, row],
        out_shape=[
            jax.ShapeDtypeStruct((_E, _D), _f32),
            jax.ShapeDtypeStruct((_E, _D), _f32),
            jax.ShapeDtypeStruct((_E, _D), _f32),
            jax.ShapeDtypeStruct((1, _D), _f32),
        ],
    )(*args)


def _tc_node(O0, O1, h, ww1, ww2, bw, wea, web, g, b, wqn=None, bqn=None):
    first = wqn is not None

    def body(*refs):
        if first:
            (o0r, o1r, hr, ww1r, ww2r, bwr, wear, webr, gr, br,
             wqr, bqr, hpr, pr, rr, qr, qmr) = refs
        else:
            (o0r, o1r, hr, ww1r, ww2r, bwr, wear, webr, gr, br,
             hpr, pr, rr) = refs
        i = pl.program_id(0)
        o0 = o0r[...]
        o1 = o1r[...]
        ss = jnp.concatenate([o0[:, :64], o1[:, :64]], axis=1)
        tt = jnp.concatenate([o0[:, 64:], o1[:, 64:]], axis=1)
        pos = ss > 0.0
        hn = jnp.where(pos, tt / jnp.where(pos, ss, 1.0), 0.0)
        h16 = hr[...].astype(_bf16)
        h_new = _dot(hn.astype(_bf16), ww1r[...]) + _dot(h16, ww2r[...]) \
            + bwr[...]
        hn16 = h_new.astype(_bf16)
        pr[...] = _dot(hn16, wear[...])
        rr[...] = _dot(hn16, webr[...])
        hp = _ln(jnp.maximum(h_new, 0.0), gr[...], br[...])
        hpr[...] = hp
        if first:
            q = _dot(hp.astype(_bf16), wqr[...]) + bqr[...]
            qr[...] = q
            bm = jnp.max(jnp.abs(q), axis=0, keepdims=True)
            prev = jnp.where(i == 0, jnp.zeros_like(bm), qmr[...])
            qmr[...] = jnp.maximum(prev, bm)

    full = pl.BlockSpec((_D, _D), lambda i: (0, 0))
    row = pl.BlockSpec((1, _D), lambda i: (0, 0))
    nb = pl.BlockSpec((_NB, _D), lambda i: (i, 0))
    in_specs = [nb, nb, nb, full, full, row, full, full, row, row]
    out_specs = [nb, nb, nb]
    out_shape = [jax.ShapeDtypeStruct((_N, _D), _f32)] * 3
    args = [O0, O1, h, ww1, ww2, bw, wea, web, g, b]
    if first:
        in_specs = in_specs + [full, row]
        out_specs = out_specs + [nb, row]
        out_shape = out_shape + [
            jax.ShapeDtypeStruct((_N, _D), _f32),
            jax.ShapeDtypeStruct((1, _D), _f32),
        ]
        args = args + [wqn, bqn]
    return pl.pallas_call(
        body,
        grid=(_N // _NB,),
        in_specs=in_specs,
        out_specs=out_specs,
        out_shape=out_shape,
    )(*args)


def _tc_final(esum, te, g, b):
    def body(e_ref, t_ref, gr, br, o_ref):
        o_ref[...] = _ln(jnp.maximum(e_ref[...] + t_ref[...], 0.0),
                         gr[...], br[...])

    eb = pl.BlockSpec((_RB, _D), lambda i: (i, 0))
    row = pl.BlockSpec((1, _D), lambda i: (0, 0))
    return pl.pallas_call(
        body,
        grid=(_E // _RB,),
        in_specs=[eb, eb, row, row],
        out_specs=eb,
        out_shape=jax.ShapeDtypeStruct((_E, _D), _f32),
    )(esum, te, g, b)


def kernel(x, edge_attr, edge_index, Wq0, bq0, Wk0, bk0, Wv0, bv0, Ww0,
           bw0, We0, be0, Wq1, bq1, Wk1, bk1, Wv1, bv1, Ww1, bw1, We1,
           be1, gamma, beta):
    ei = edge_index.reshape(2, _NCHUNK, 1, _CW)
    src2d, dst2d = ei[0], ei[1]
    zrows = jnp.zeros((624, _D), _f32)
    g = gamma.reshape(1, _D)
    b = beta.reshape(1, _D)

    def w16(w):
        return w.astype(_bf16)

    q, qmax = _tc_qpass(x, w16(Wq0), bq0.reshape(1, _D))
    hs = _sc_gather(x, src2d)
    kv0, kv1, te, kmax = _tc_edge(
        hs, edge_attr, w16(Wk0[:128]), w16(Wk0[128:]), w16(Wv0[:128]),
        w16(Wv0[128:]), w16(We0[256:]), bk0.reshape(1, _D),
        bv0.reshape(1, _D), be0.reshape(1, _D), g, b, second=False)
    o0, o1 = _sc_attn(kv0, kv1, q, qmax * kmax, dst2d, zrows)
    h1, p1, r1, q, qmax = _tc_node(
        o0, o1, x, w16(Ww0[:128]), w16(Ww0[128:]),
        bw0.reshape(1, _D), w16(We0[:128]), w16(We0[128:256]), g, b,
        w16(Wq1), bq1.reshape(1, _D))
    esum1, hs2 = _sc_edgec(p1, r1, h1, src2d, dst2d)

    kv0, kv1, te, kmax = _tc_edge(
        hs2, esum1, w16(Wk1[:128]), w16(Wk1[128:]), w16(Wv1[:128]),
        w16(Wv1[128:]), w16(We1[256:]), bk1.reshape(1, _D),
        bv1.reshape(1, _D), be1.reshape(1, _D), g, b, second=True,
        tprev=te)
    o0, o1 = _sc_attn(kv0, kv1, q, qmax * kmax, dst2d, zrows)
    h2, p2, r2 = _tc_node(
        o0, o1, h1, w16(Ww1[:128]), w16(Ww1[128:]),
        bw1.reshape(1, _D), w16(We1[:128]), w16(We1[128:256]), g, b)
    (esum2,) = _sc_edgec(p2, r2, None, src2d, dst2d)
    e_out = _tc_final(esum2, te, g, b)
    return h2, e_out


# speedup vs baseline: 6.6328x; 1.0999x over previous
import functools

import jax
import jax.numpy as jnp
from jax import lax
from jax.experimental import pallas as pl
from jax.experimental.pallas import tpu as pltpu
from jax.experimental.pallas import tpu_sc as plsc

_N = 10000
_E = 320000
_D = 128
_CW = 128
_NCHUNK = _E // _CW
_RB = 4000
_NB = 400
_NPAD = 10240

_f32 = jnp.float32
_bf16 = jnp.bfloat16


def _mesh():
    return plsc.VectorSubcoreMesh(core_axis_name="c", subcore_axis_name="s")


def _ln(v, g, b):
    mu = jnp.mean(v, axis=-1, keepdims=True)
    var = jnp.mean((v - mu) ** 2, axis=-1, keepdims=True)
    return (v - mu) * lax.rsqrt(var + 1e-5) * g + b


def _dot(a16, w16):
    return jnp.dot(a16, w16, preferred_element_type=_f32)


def _sc_gather(table, src2d):
    @functools.partial(
        pl.kernel,
        out_type=jax.ShapeDtypeStruct((_E, _D), _f32),
        mesh=_mesh(),
        scratch_types=[
            pltpu.VMEM((79, 1, _CW), jnp.int32),
            pltpu.VMEM((_CW, _D), _f32),
            pltpu.VMEM((_CW, _D), _f32),
            pltpu.SemaphoreType.DMA,
            pltpu.SemaphoreType.DMA,
        ],
    )
    def k(table_hbm, idx_hbm, out_hbm, idx_a, buf0, buf1, sem0, sem1):
        wid = lax.axis_index("c") * 16 + lax.axis_index("s")
        nk = jnp.where(wid < 4, 79, 78)
        j0 = wid * 78 + jnp.minimum(wid, 4)
        pltpu.sync_copy(idx_hbm.at[pl.ds(j0, 78)], idx_a.at[pl.ds(0, 78)])

        @pl.when(wid < 4)
        def _():
            pltpu.sync_copy(idx_hbm.at[j0 + 78], idx_a.at[78])

        pltpu.async_copy(table_hbm.at[idx_a.at[0, 0]], buf0, sem0)

        def body(kk, carry):
            even = kk % 2 == 0

            def step(buf, sem, obuf, osem):
                @pl.when(kk + 1 < nk)
                def _():
                    pltpu.async_copy(
                        table_hbm.at[idx_a.at[kk + 1, 0]], obuf, osem)
                pltpu.make_async_copy(
                    table_hbm.at[idx_a.at[kk, 0]], buf, sem).wait()
                pltpu.sync_copy(buf, out_hbm.at[pl.ds((j0 + kk) * _CW, _CW)])

            @pl.when(even)
            def _():
                step(buf0, sem0, buf1, sem1)

            @pl.when(jnp.logical_not(even))
            def _():
                step(buf1, sem1, buf0, sem0)

            return carry

        lax.fori_loop(0, nk, body, 0)

    return k(table, src2d)


def _sc_attn(KV0, KV1, Q, M, dst2d, zrows):
    out_t = [jax.ShapeDtypeStruct((_N, _D), _f32)] * 2

    @functools.partial(
        pl.kernel,
        out_type=out_t,
        mesh=_mesh(),
        scratch_types=[
            pltpu.VMEM((1, _CW), jnp.int32),
            pltpu.VMEM((1, _CW), jnp.int32),
            pltpu.VMEM((_CW, _D), _f32),
            pltpu.VMEM((_CW, _D), _f32),
            pltpu.VMEM((_CW, _D), _f32),
            pltpu.VMEM((1, _D), _f32),
            pltpu.VMEM_SHARED((_N, _D), _f32),
            pltpu.SemaphoreType.DMA,
            pltpu.SemaphoreType.DMA,
            pltpu.SemaphoreType.DMA,
            pltpu.SemaphoreType.DMA,
            pltpu.SemaphoreType.DMA,
        ],
    )
    def k(kv0h, kv1h, qh, mh, dsth, zh, o0, o1,
          idx0, idx1, kv0v, kv1v, qd_v, m_v, acc,
          si0, si1, sk0, sk1, sq):
        c = lax.axis_index("c")
        s = lax.axis_index("s")
        pltpu.sync_copy(zh, acc.at[pl.ds(s * 624, 624)])

        @pl.when(s == 15)
        def _():
            pltpu.sync_copy(zh.at[pl.ds(0, 16)], acc.at[pl.ds(9984, 16)])

        plsc.subcore_barrier()
        nk = jnp.where(s < 4, 157, 156)
        j0 = s * 156 + jnp.minimum(s, 4)
        pltpu.sync_copy(mh, m_v)

        def jrow(kk):
            return jnp.minimum(j0 + kk, _NCHUNK - 1)

        def run(kvh, coff):
            ms = [m_v[pl.ds(0, 1), pl.ds(coff + 16 * cc, 16)]
                  for cc in range(4)]

            def issue_idx(kk, ib, si):
                pltpu.async_copy(dsth.at[jrow(kk)], ib, si)

            def issue_kv(kk, kvb, sk):
                pltpu.async_copy(
                    kvh.at[pl.ds(jrow(kk) * _CW, _CW)], kvb, sk)

            issue_idx(0, idx0, si0)
            pltpu.make_async_copy(dsth.at[jrow(0)], idx0, si0).wait()
            issue_kv(0, kv0v, sk0)
            pltpu.async_copy(qh.at[idx0.at[0]], qd_v, sq)
            issue_idx(1, idx1, si1)

            def step(kk, ib, si, oib, osi, kvb, sk, okvb, osk):
                @pl.when(kk + 1 < nk)
                def _():
                    issue_kv(kk + 1, okvb, osk)

                pltpu.make_async_copy(
                    kvh.at[pl.ds(jrow(kk) * _CW, _CW)], kvb, sk).wait()
                pltpu.make_async_copy(qh.at[ib.at[0]], qd_v, sq).wait()

                @pl.loop(0, _CW, step=4)
                def _(r0):
                    for dr in range(4):
                        r = r0 + dr
                        for cc in range(4):
                            sl = (pl.ds(r, 1), pl.ds(16 * cc, 16))
                            slq = (pl.ds(r, 1),
                                   pl.ds(coff + 16 * cc, 16))
                            slv = (pl.ds(r, 1),
                                   pl.ds(64 + 16 * cc, 16))
                            a = jnp.exp(qd_v[slq] * kvb[sl] - ms[cc])
                            kvb[sl] = a
                            kvb[slv] = a * kvb[slv]

                @pl.when(kk + 1 < nk)
                def _():
                    pltpu.make_async_copy(dsth.at[jrow(kk + 1)], oib,
                                          osi).wait()
                    pltpu.async_copy(qh.at[oib.at[0]], qd_v, sq)

                pltpu.sync_copy(kvb, acc.at[ib.at[0]], add=True)

                @pl.when(kk + 2 < nk)
                def _():
                    issue_idx(kk + 2, ib, si)

            def body(kk, carry):
                @pl.when(kk % 2 == 0)
                def _():
                    step(kk, idx0, si0, idx1, si1, kv0v, sk0, kv1v, sk1)

                @pl.when(kk % 2 == 1)
                def _():
                    step(kk, idx1, si1, idx0, si0, kv1v, sk1, kv0v, sk0)

                return carry

            lax.fori_loop(0, nk, body, 0)

        @pl.when(c == 0)
        def _():
            run(kv0h, 0)

        @pl.when(c == 1)
        def _():
            run(kv1h, 64)

        plsc.subcore_barrier()

        def writeout(oo):
            pltpu.sync_copy(acc.at[pl.ds(s * 624, 624)],
                            oo.at[pl.ds(s * 624, 624)])

            @pl.when(s == 15)
            def _():
                pltpu.sync_copy(acc.at[pl.ds(9984, 16)],
                                oo.at[pl.ds(9984, 16)])

        @pl.when(c == 0)
        def _():
            writeout(o0)

        @pl.when(c == 1)
        def _():
            writeout(o1)

    return k(KV0, KV1, Q, M, dst2d, zrows)


def _sc_edgec(P, R, hpost, src2d, dst2d):
    with_h = hpost is not None
    out_t = [jax.ShapeDtypeStruct((_E, _D), _f32)]
    scratch = [
        pltpu.VMEM((79, 1, _CW), jnp.int32),
        pltpu.VMEM((79, 1, _CW), jnp.int32),
        pltpu.VMEM((_CW, _D), _f32),
        pltpu.VMEM((_CW, _D), _f32),
        pltpu.VMEM((_CW, _D), _f32),
        pltpu.VMEM((_CW, _D), _f32),
        pltpu.SemaphoreType.DMA,
        pltpu.SemaphoreType.DMA,
        pltpu.SemaphoreType.DMA,
        pltpu.SemaphoreType.DMA,
    ]
    if with_h:
        out_t = out_t + [jax.ShapeDtypeStruct((_E, _D), _f32)]
        scratch = scratch + [pltpu.VMEM((_CW, _D), _f32),
                             pltpu.VMEM((_CW, _D), _f32),
                             pltpu.SemaphoreType.DMA,
                             pltpu.SemaphoreType.DMA]

    def body_fn(ph, rh, *rest):
        if with_h:
            (hh, srch, dsth, eo, ho, idxs_a, idxd_a, pb0, pb1, rb0, rb1,
             sp0, sp1, sr0, sr1, hb0, hb1, sh0, sh1) = rest
        else:
            (srch, dsth, eo, idxs_a, idxd_a, pb0, pb1, rb0, rb1,
             sp0, sp1, sr0, sr1) = rest
        wid = lax.axis_index("c") * 16 + lax.axis_index("s")
        nk = jnp.where(wid < 4, 79, 78)
        j0 = wid * 78 + jnp.minimum(wid, 4)
        pltpu.sync_copy(srch.at[pl.ds(j0, 78)], idxs_a.at[pl.ds(0, 78)])
        pltpu.sync_copy(dsth.at[pl.ds(j0, 78)], idxd_a.at[pl.ds(0, 78)])

        @pl.when(wid < 4)
        def _():
            pltpu.sync_copy(srch.at[j0 + 78], idxs_a.at[78])
            pltpu.sync_copy(dsth.at[j0 + 78], idxd_a.at[78])

        def loads(kk, pb, rb, hb, sp, sr, sh):
            pltpu.async_copy(ph.at[idxs_a.at[kk, 0]], pb, sp)
            pltpu.async_copy(rh.at[idxd_a.at[kk, 0]], rb, sr)
            if with_h:
                pltpu.async_copy(hh.at[idxs_a.at[kk, 0]], hb, sh)

        def step(kk, pb, rb, hb, sp, sr, sh, opb, orb, ohb, osp, osr, osh):
            @pl.when(kk + 1 < nk)
            def _():
                loads(kk + 1, opb, orb, ohb, osp, osr, osh)

            pltpu.make_async_copy(ph.at[idxs_a.at[kk, 0]], pb, sp).wait()
            pltpu.make_async_copy(rh.at[idxd_a.at[kk, 0]], rb, sr).wait()
            if with_h:
                pltpu.make_async_copy(hh.at[idxs_a.at[kk, 0]], hb,
                                      sh).wait()
                pltpu.sync_copy(hb, ho.at[pl.ds((j0 + kk) * _CW, _CW)])

            @pl.loop(0, _CW, step=4)
            def _(r0):
                for dr in range(4):
                    for cc in range(8):
                        sl = (pl.ds(r0 + dr, 1), pl.ds(16 * cc, 16))
                        pb[sl] = pb[sl] + rb[sl]

            pltpu.sync_copy(pb, eo.at[pl.ds((j0 + kk) * _CW, _CW)])

        if with_h:
            loads(0, pb0, rb0, hb0, sp0, sr0, sh0)
        else:
            loads(0, pb0, rb0, None, sp0, sr0, None)

        def body(kk, carry):
            if with_h:
                @pl.when(kk % 2 == 0)
                def _():
                    step(kk, pb0, rb0, hb0, sp0, sr0, sh0,
                         pb1, rb1, hb1, sp1, sr1, sh1)

                @pl.when(kk % 2 == 1)
                def _():
                    step(kk, pb1, rb1, hb1, sp1, sr1, sh1,
                         pb0, rb0, hb0, sp0, sr0, sh0)
            else:
                @pl.when(kk % 2 == 0)
                def _():
                    step(kk, pb0, rb0, None, sp0, sr0, None,
                         pb1, rb1, None, sp1, sr1, None)

                @pl.when(kk % 2 == 1)
                def _():
                    step(kk, pb1, rb1, None, sp1, sr1, None,
                         pb0, rb0, None, sp0, sr0, None)
            return carry

        lax.fori_loop(0, nk, body, 0)

    k = functools.partial(pl.kernel, out_type=out_t, mesh=_mesh(),
                          scratch_types=scratch)(body_fn)
    if with_h:
        return k(P, R, hpost, src2d, dst2d)
    return k(P, R, src2d, dst2d)


def _tc_qpass(h, wq16, bq):
    def body(h_ref, w_ref, b_ref, q_ref, qm_ref):
        i = pl.program_id(0)
        q = _dot(h_ref[...].astype(_bf16), w_ref[...]) + b_ref[...]
        q_ref[...] = q
        bm = jnp.max(jnp.abs(q), axis=0, keepdims=True)
        prev = jnp.where(i == 0, jnp.zeros_like(bm), qm_ref[...])
        qm_ref[...] = jnp.maximum(prev, bm)

    return pl.pallas_call(
        body,
        grid=(_N // _NB,),
        in_specs=[
            pl.BlockSpec((_NB, _D), lambda i: (i, 0)),
            pl.BlockSpec((_D, _D), lambda i: (0, 0)),
            pl.BlockSpec((1, _D), lambda i: (0, 0)),
        ],
        out_specs=[
            pl.BlockSpec((_NB, _D), lambda i: (i, 0)),
            pl.BlockSpec((1, _D), lambda i: (0, 0)),
        ],
        out_shape=[
            jax.ShapeDtypeStruct((_N, _D), _f32),
            jax.ShapeDtypeStruct((1, _D), _f32),
        ],
    )(h, wq16, bq)


def _tc_edge(hs, ein, wk1, wk2, wv1, wv2, wec, bk, bv, be, g, b, second,
             tprev=None):
    def body(hs_ref, e_ref, *refs):
        if second:
            (tpr, wk1r, wk2r, wv1r, wv2r, wecr, bkr, bvr, ber,
             gr, br, kv0r, kv1r, ter, kmr) = refs
        else:
            (wk1r, wk2r, wv1r, wv2r, wecr, bkr, bvr, ber,
             gr, br, kv0r, kv1r, ter, kmr) = refs
        i = pl.program_id(0)
        e_blk = e_ref[...]
        if second:
            e_blk = _ln(jnp.maximum(e_blk + tpr[...], 0.0), gr[...],
                        br[...])
        hsb = hs_ref[...]
        st16 = (hsb * e_blk).astype(_bf16)
        hs16 = hsb.astype(_bf16)
        e16 = e_blk.astype(_bf16)
        kk = _dot(st16, wk1r[...]) + _dot(hs16, wk2r[...]) + bkr[...]
        vv = _dot(st16, wv1r[...]) + _dot(hs16, wv2r[...]) + bvr[...]
        ter[...] = _dot(e16, wecr[...]) + ber[...]
        kv0r[...] = jnp.concatenate([kk[:, :64], vv[:, :64]], axis=1)
        kv1r[...] = jnp.concatenate([kk[:, 64:], vv[:, 64:]], axis=1)
        bm = jnp.max(jnp.abs(kk), axis=0, keepdims=True)
        prev = jnp.where(i == 0, jnp.zeros_like(bm), kmr[...])
        kmr[...] = jnp.maximum(prev, bm)

    full = pl.BlockSpec((_D, _D), lambda i: (0, 0))
    row = pl.BlockSpec((1, _D), lambda i: (0, 0))
    eb = pl.BlockSpec((_RB, _D), lambda i: (i, 0))
    in_specs = [eb, eb]
    args = [hs, ein]
    if second:
        in_specs = in_specs + [eb]
        args = args + [tprev]
    in_specs = in_specs + [full, full, full, full, full, row, row, row,
                           row, row]
    args = args + [wk1, wk2, wv1, wv2, wec, bk, bv, be, g, b]
    return pl.pallas_call(
        body,
        grid=(_E // _RB,),
        in_specs=in_specs,
        out_specs=[eb, eb, eb, row],
        out_shape=[
            jax.ShapeDtypeStruct((_E, _D), _f32),
            jax.ShapeDtypeStruct((_E, _D), _f32),
            jax.ShapeDtypeStruct((_E, _D), _f32),
            jax.ShapeDtypeStruct((1, _D), _f32),
        ],
    )(*args)


def _tc_node(O0, O1, h, ww1, ww2, bw, wea, web, g, b, wqn=None, bqn=None):
    first = wqn is not None

    def body(*refs):
        if first:
            (o0r, o1r, hr, ww1r, ww2r, bwr, wear, webr, gr, br,
             wqr, bqr, hpr, pr, rr, qr, qmr) = refs
        else:
            (o0r, o1r, hr, ww1r, ww2r, bwr, wear, webr, gr, br,
             hpr, pr, rr) = refs
        i = pl.program_id(0)
        o0 = o0r[...]
        o1 = o1r[...]
        ss = jnp.concatenate([o0[:, :64], o1[:, :64]], axis=1)
        tt = jnp.concatenate([o0[:, 64:], o1[:, 64:]], axis=1)
        pos = ss > 0.0
        hn = jnp.where(pos, tt / jnp.where(pos, ss, 1.0), 0.0)
        h16 = hr[...].astype(_bf16)
        h_new = _dot(hn.astype(_bf16), ww1r[...]) + _dot(h16, ww2r[...]) \
            + bwr[...]
        hn16 = h_new.astype(_bf16)
        pr[...] = _dot(hn16, wear[...])
        rr[...] = _dot(hn16, webr[...])
        hp = _ln(jnp.maximum(h_new, 0.0), gr[...], br[...])
        hpr[...] = hp
        if first:
            q = _dot(hp.astype(_bf16), wqr[...]) + bqr[...]
            qr[...] = q
            bm = jnp.max(jnp.abs(q), axis=0, keepdims=True)
            prev = jnp.where(i == 0, jnp.zeros_like(bm), qmr[...])
            qmr[...] = jnp.maximum(prev, bm)

    full = pl.BlockSpec((_D, _D), lambda i: (0, 0))
    row = pl.BlockSpec((1, _D), lambda i: (0, 0))
    nb = pl.BlockSpec((_NB, _D), lambda i: (i, 0))
    in_specs = [nb, nb, nb, full, full, row, full, full, row, row]
    out_specs = [nb, nb, nb]
    out_shape = [jax.ShapeDtypeStruct((_N, _D), _f32)] * 3
    args = [O0, O1, h, ww1, ww2, bw, wea, web, g, b]
    if first:
        in_specs = in_specs + [full, row]
        out_specs = out_specs + [nb, row]
        out_shape = out_shape + [
            jax.ShapeDtypeStruct((_N, _D), _f32),
            jax.ShapeDtypeStruct((1, _D), _f32),
        ]
        args = args + [wqn, bqn]
    return pl.pallas_call(
        body,
        grid=(_N // _NB,),
        in_specs=in_specs,
        out_specs=out_specs,
        out_shape=out_shape,
    )(*args)


def _tc_final(esum, te, g, b):
    def body(e_ref, t_ref, gr, br, o_ref):
        o_ref[...] = _ln(jnp.maximum(e_ref[...] + t_ref[...], 0.0),
                         gr[...], br[...])

    eb = pl.BlockSpec((_RB, _D), lambda i: (i, 0))
    row = pl.BlockSpec((1, _D), lambda i: (0, 0))
    return pl.pallas_call(
        body,
        grid=(_E // _RB,),
        in_specs=[eb, eb, row, row],
        out_specs=eb,
        out_shape=jax.ShapeDtypeStruct((_E, _D), _f32),
    )(esum, te, g, b)


def kernel(x, edge_attr, edge_index, Wq0, bq0, Wk0, bk0, Wv0, bv0, Ww0,
           bw0, We0, be0, Wq1, bq1, Wk1, bk1, Wv1, bv1, Ww1, bw1, We1,
           be1, gamma, beta):
    ei = edge_index.reshape(2, _NCHUNK, 1, _CW)
    src2d, dst2d = ei[0], ei[1]
    zrows = jnp.zeros((624, _D), _f32)
    g = gamma.reshape(1, _D)
    b = beta.reshape(1, _D)

    def w16(w):
        return w.astype(_bf16)

    q, qmax = _tc_qpass(x, w16(Wq0), bq0.reshape(1, _D))
    hs = _sc_gather(x, src2d)
    kv0, kv1, te, kmax = _tc_edge(
        hs, edge_attr, w16(Wk0[:128]), w16(Wk0[128:]), w16(Wv0[:128]),
        w16(Wv0[128:]), w16(We0[256:]), bk0.reshape(1, _D),
        bv0.reshape(1, _D), be0.reshape(1, _D), g, b, second=False)
    o0, o1 = _sc_attn(kv0, kv1, q, qmax * kmax, dst2d, zrows)
    h1, p1, r1, q, qmax = _tc_node(
        o0, o1, x, w16(Ww0[:128]), w16(Ww0[128:]),
        bw0.reshape(1, _D), w16(We0[:128]), w16(We0[128:256]), g, b,
        w16(Wq1), bq1.reshape(1, _D))
    esum1, hs2 = _sc_edgec(p1, r1, h1, src2d, dst2d)

    kv0, kv1, te, kmax = _tc_edge(
        hs2, esum1, w16(Wk1[:128]), w16(Wk1[128:]), w16(Wv1[:128]),
        w16(Wv1[128:]), w16(We1[256:]), bk1.reshape(1, _D),
        bv1.reshape(1, _D), be1.reshape(1, _D), g, b, second=True,
        tprev=te)
    o0, o1 = _sc_attn(kv0, kv1, q, qmax * kmax, dst2d, zrows)
    h2, p2, r2 = _tc_node(
        o0, o1, h1, w16(Ww1[:128]), w16(Ww1[128:]),
        bw1.reshape(1, _D), w16(We1[:128]), w16(We1[128:256]), g, b)
    (esum2,) = _sc_edgec(p2, r2, None, src2d, dst2d)
    e_out = _tc_final(esum2, te, g, b)
    return h2, e_out


# speedup vs baseline: 6.6789x; 1.0069x over previous
import functools

import jax
import jax.numpy as jnp
from jax import lax
from jax.experimental import pallas as pl
from jax.experimental.pallas import tpu as pltpu
from jax.experimental.pallas import tpu_sc as plsc

_N = 10000
_E = 320000
_D = 128
_CW = 128
_NCHUNK = _E // _CW
_RB = 4000
_NB = 2000
_NPAD = 10240

_f32 = jnp.float32
_bf16 = jnp.bfloat16


def _mesh():
    return plsc.VectorSubcoreMesh(core_axis_name="c", subcore_axis_name="s")


def _ln(v, g, b):
    mu = jnp.mean(v, axis=-1, keepdims=True)
    var = jnp.mean((v - mu) ** 2, axis=-1, keepdims=True)
    return (v - mu) * lax.rsqrt(var + 1e-5) * g + b


def _dot(a16, w16):
    return jnp.dot(a16, w16, preferred_element_type=_f32)


def _sc_gather(table, src2d):
    @functools.partial(
        pl.kernel,
        out_type=jax.ShapeDtypeStruct((_E, _D), _f32),
        mesh=_mesh(),
        scratch_types=[
            pltpu.VMEM((79, 1, _CW), jnp.int32),
            pltpu.VMEM((_CW, _D), _f32),
            pltpu.VMEM((_CW, _D), _f32),
            pltpu.SemaphoreType.DMA,
            pltpu.SemaphoreType.DMA,
        ],
    )
    def k(table_hbm, idx_hbm, out_hbm, idx_a, buf0, buf1, sem0, sem1):
        wid = lax.axis_index("c") * 16 + lax.axis_index("s")
        nk = jnp.where(wid < 4, 79, 78)
        j0 = wid * 78 + jnp.minimum(wid, 4)
        pltpu.sync_copy(idx_hbm.at[pl.ds(j0, 78)], idx_a.at[pl.ds(0, 78)])

        @pl.when(wid < 4)
        def _():
            pltpu.sync_copy(idx_hbm.at[j0 + 78], idx_a.at[78])

        pltpu.async_copy(table_hbm.at[idx_a.at[0, 0]], buf0, sem0)

        def body(kk, carry):
            even = kk % 2 == 0

            def step(buf, sem, obuf, osem):
                @pl.when(kk + 1 < nk)
                def _():
                    pltpu.async_copy(
                        table_hbm.at[idx_a.at[kk + 1, 0]], obuf, osem)
                pltpu.make_async_copy(
                    table_hbm.at[idx_a.at[kk, 0]], buf, sem).wait()
                pltpu.sync_copy(buf, out_hbm.at[pl.ds((j0 + kk) * _CW, _CW)])

            @pl.when(even)
            def _():
                step(buf0, sem0, buf1, sem1)

            @pl.when(jnp.logical_not(even))
            def _():
                step(buf1, sem1, buf0, sem0)

            return carry

        lax.fori_loop(0, nk, body, 0)

    return k(table, src2d)


def _sc_attn(KV0, KV1, Q, M, dst2d, zrows):
    out_t = [jax.ShapeDtypeStruct((_N, _D), _f32)] * 2

    @functools.partial(
        pl.kernel,
        out_type=out_t,
        mesh=_mesh(),
        scratch_types=[
            pltpu.VMEM((1, _CW), jnp.int32),
            pltpu.VMEM((1, _CW), jnp.int32),
            pltpu.VMEM((_CW, _D), _f32),
            pltpu.VMEM((_CW, _D), _f32),
            pltpu.VMEM((_CW, _D), _f32),
            pltpu.VMEM((1, _D), _f32),
            pltpu.VMEM_SHARED((_N, _D), _f32),
            pltpu.SemaphoreType.DMA,
            pltpu.SemaphoreType.DMA,
            pltpu.SemaphoreType.DMA,
            pltpu.SemaphoreType.DMA,
            pltpu.SemaphoreType.DMA,
        ],
    )
    def k(kv0h, kv1h, qh, mh, dsth, zh, o0, o1,
          idx0, idx1, kv0v, kv1v, qd_v, m_v, acc,
          si0, si1, sk0, sk1, sq):
        c = lax.axis_index("c")
        s = lax.axis_index("s")
        pltpu.sync_copy(zh, acc.at[pl.ds(s * 624, 624)])

        @pl.when(s == 15)
        def _():
            pltpu.sync_copy(zh.at[pl.ds(0, 16)], acc.at[pl.ds(9984, 16)])

        plsc.subcore_barrier()
        nk = jnp.where(s < 4, 157, 156)
        j0 = s * 156 + jnp.minimum(s, 4)
        pltpu.sync_copy(mh, m_v)

        def jrow(kk):
            return jnp.minimum(j0 + kk, _NCHUNK - 1)

        def run(kvh, coff):
            ms = [m_v[pl.ds(0, 1), pl.ds(coff + 16 * cc, 16)]
                  for cc in range(4)]

            def issue_idx(kk, ib, si):
                pltpu.async_copy(dsth.at[jrow(kk)], ib, si)

            def issue_kv(kk, kvb, sk):
                pltpu.async_copy(
                    kvh.at[pl.ds(jrow(kk) * _CW, _CW)], kvb, sk)

            issue_idx(0, idx0, si0)
            pltpu.make_async_copy(dsth.at[jrow(0)], idx0, si0).wait()
            issue_kv(0, kv0v, sk0)
            pltpu.async_copy(qh.at[idx0.at[0]], qd_v, sq)
            issue_idx(1, idx1, si1)

            def step(kk, ib, si, oib, osi, kvb, sk, okvb, osk):
                @pl.when(kk + 1 < nk)
                def _():
                    issue_kv(kk + 1, okvb, osk)

                pltpu.make_async_copy(
                    kvh.at[pl.ds(jrow(kk) * _CW, _CW)], kvb, sk).wait()
                pltpu.make_async_copy(qh.at[ib.at[0]], qd_v, sq).wait()

                @pl.loop(0, _CW, step=8)
                def _(r0):
                    for dr in range(8):
                        r = r0 + dr
                        for cc in range(4):
                            sl = (pl.ds(r, 1), pl.ds(16 * cc, 16))
                            slq = (pl.ds(r, 1),
                                   pl.ds(coff + 16 * cc, 16))
                            slv = (pl.ds(r, 1),
                                   pl.ds(64 + 16 * cc, 16))
                            a = jnp.exp(qd_v[slq] * kvb[sl] - ms[cc])
                            kvb[sl] = a
                            kvb[slv] = a * kvb[slv]

                @pl.when(kk + 1 < nk)
                def _():
                    pltpu.make_async_copy(dsth.at[jrow(kk + 1)], oib,
                                          osi).wait()
                    pltpu.async_copy(qh.at[oib.at[0]], qd_v, sq)

                pltpu.sync_copy(kvb, acc.at[ib.at[0]], add=True)

                @pl.when(kk + 2 < nk)
                def _():
                    issue_idx(kk + 2, ib, si)

            def body(kk, carry):
                @pl.when(kk % 2 == 0)
                def _():
                    step(kk, idx0, si0, idx1, si1, kv0v, sk0, kv1v, sk1)

                @pl.when(kk % 2 == 1)
                def _():
                    step(kk, idx1, si1, idx0, si0, kv1v, sk1, kv0v, sk0)

                return carry

            lax.fori_loop(0, nk, body, 0)

        @pl.when(c == 0)
        def _():
            run(kv0h, 0)

        @pl.when(c == 1)
        def _():
            run(kv1h, 64)

        plsc.subcore_barrier()

        def writeout(oo):
            pltpu.sync_copy(acc.at[pl.ds(s * 624, 624)],
                            oo.at[pl.ds(s * 624, 624)])

            @pl.when(s == 15)
            def _():
                pltpu.sync_copy(acc.at[pl.ds(9984, 16)],
                                oo.at[pl.ds(9984, 16)])

        @pl.when(c == 0)
        def _():
            writeout(o0)

        @pl.when(c == 1)
        def _():
            writeout(o1)

    return k(KV0, KV1, Q, M, dst2d, zrows)


def _sc_edgec(P, R, hpost, src2d, dst2d):
    with_h = hpost is not None
    out_t = [jax.ShapeDtypeStruct((_E, _D), _f32)]
    scratch = [
        pltpu.VMEM((79, 1, _CW), jnp.int32),
        pltpu.VMEM((79, 1, _CW), jnp.int32),
        pltpu.VMEM((_CW, _D), _f32),
        pltpu.VMEM((_CW, _D), _f32),
        pltpu.VMEM((_CW, _D), _f32),
        pltpu.VMEM((_CW, _D), _f32),
        pltpu.SemaphoreType.DMA,
        pltpu.SemaphoreType.DMA,
        pltpu.SemaphoreType.DMA,
        pltpu.SemaphoreType.DMA,
    ]
    if with_h:
        out_t = out_t + [jax.ShapeDtypeStruct((_E, _D), _f32)]
        scratch = scratch + [pltpu.VMEM((_CW, _D), _f32),
                             pltpu.VMEM((_CW, _D), _f32),
                             pltpu.SemaphoreType.DMA,
                             pltpu.SemaphoreType.DMA]

    def body_fn(ph, rh, *rest):
        if with_h:
            (hh, srch, dsth, eo, ho, idxs_a, idxd_a, pb0, pb1, rb0, rb1,
             sp0, sp1, sr0, sr1, hb0, hb1, sh0, sh1) = rest
        else:
            (srch, dsth, eo, idxs_a, idxd_a, pb0, pb1, rb0, rb1,
             sp0, sp1, sr0, sr1) = rest
        wid = lax.axis_index("c") * 16 + lax.axis_index("s")
        nk = jnp.where(wid < 4, 79, 78)
        j0 = wid * 78 + jnp.minimum(wid, 4)
        pltpu.sync_copy(srch.at[pl.ds(j0, 78)], idxs_a.at[pl.ds(0, 78)])
        pltpu.sync_copy(dsth.at[pl.ds(j0, 78)], idxd_a.at[pl.ds(0, 78)])

        @pl.when(wid < 4)
        def _():
            pltpu.sync_copy(srch.at[j0 + 78], idxs_a.at[78])
            pltpu.sync_copy(dsth.at[j0 + 78], idxd_a.at[78])

        def loads(kk, pb, rb, hb, sp, sr, sh):
            pltpu.async_copy(ph.at[idxs_a.at[kk, 0]], pb, sp)
            pltpu.async_copy(rh.at[idxd_a.at[kk, 0]], rb, sr)
            if with_h:
                pltpu.async_copy(hh.at[idxs_a.at[kk, 0]], hb, sh)

        def step(kk, pb, rb, hb, sp, sr, sh, opb, orb, ohb, osp, osr, osh):
            @pl.when(kk + 1 < nk)
            def _():
                loads(kk + 1, opb, orb, ohb, osp, osr, osh)

            pltpu.make_async_copy(ph.at[idxs_a.at[kk, 0]], pb, sp).wait()
            pltpu.make_async_copy(rh.at[idxd_a.at[kk, 0]], rb, sr).wait()
            if with_h:
                pltpu.make_async_copy(hh.at[idxs_a.at[kk, 0]], hb,
                                      sh).wait()
                pltpu.sync_copy(hb, ho.at[pl.ds((j0 + kk) * _CW, _CW)])

            @pl.loop(0, _CW, step=4)
            def _(r0):
                for dr in range(4):
                    for cc in range(8):
                        sl = (pl.ds(r0 + dr, 1), pl.ds(16 * cc, 16))
                        pb[sl] = pb[sl] + rb[sl]

            pltpu.sync_copy(pb, eo.at[pl.ds((j0 + kk) * _CW, _CW)])

        if with_h:
            loads(0, pb0, rb0, hb0, sp0, sr0, sh0)
        else:
            loads(0, pb0, rb0, None, sp0, sr0, None)

        def body(kk, carry):
            if with_h:
                @pl.when(kk % 2 == 0)
                def _():
                    step(kk, pb0, rb0, hb0, sp0, sr0, sh0,
                         pb1, rb1, hb1, sp1, sr1, sh1)

                @pl.when(kk % 2 == 1)
                def _():
                    step(kk, pb1, rb1, hb1, sp1, sr1, sh1,
                         pb0, rb0, hb0, sp0, sr0, sh0)
            else:
                @pl.when(kk % 2 == 0)
                def _():
                    step(kk, pb0, rb0, None, sp0, sr0, None,
                         pb1, rb1, None, sp1, sr1, None)

                @pl.when(kk % 2 == 1)
                def _():
                    step(kk, pb1, rb1, None, sp1, sr1, None,
                         pb0, rb0, None, sp0, sr0, None)
            return carry

        lax.fori_loop(0, nk, body, 0)

    k = functools.partial(pl.kernel, out_type=out_t, mesh=_mesh(),
                          scratch_types=scratch)(body_fn)
    if with_h:
        return k(P, R, hpost, src2d, dst2d)
    return k(P, R, src2d, dst2d)


def _tc_qpass(h, wq16, bq):
    def body(h_ref, w_ref, b_ref, q_ref, qm_ref):
        i = pl.program_id(0)
        q = _dot(h_ref[...].astype(_bf16), w_ref[...]) + b_ref[...]
        q_ref[...] = q
        bm = jnp.max(jnp.abs(q), axis=0, keepdims=True)
        prev = jnp.where(i == 0, jnp.zeros_like(bm), qm_ref[...])
        qm_ref[...] = jnp.maximum(prev, bm)

    return pl.pallas_call(
        body,
        grid=(_N // _NB,),
        in_specs=[
            pl.BlockSpec((_NB, _D), lambda i: (i, 0)),
            pl.BlockSpec((_D, _D), lambda i: (0, 0)),
            pl.BlockSpec((1, _D), lambda i: (0, 0)),
        ],
        out_specs=[
            pl.BlockSpec((_NB, _D), lambda i: (i, 0)),
            pl.BlockSpec((1, _D), lambda i: (0, 0)),
        ],
        out_shape=[
            jax.ShapeDtypeStruct((_N, _D), _f32),
            jax.ShapeDtypeStruct((1, _D), _f32),
        ],
    )(h, wq16, bq)


def _tc_edge(hs, ein, wk1, wk2, wv1, wv2, wec, bk, bv, be, g, b, second,
             tprev=None):
    def body(hs_ref, e_ref, *refs):
        if second:
            (tpr, wk1r, wk2r, wv1r, wv2r, wecr, bkr, bvr, ber,
             gr, br, kv0r, kv1r, ter, kmr) = refs
        else:
            (wk1r, wk2r, wv1r, wv2r, wecr, bkr, bvr, ber,
             gr, br, kv0r, kv1r, ter, kmr) = refs
        i = pl.program_id(0)
        e_blk = e_ref[...]
        if second:
            e_blk = _ln(jnp.maximum(e_blk + tpr[...], 0.0), gr[...],
                        br[...])
        hsb = hs_ref[...]
        st16 = (hsb * e_blk).astype(_bf16)
        hs16 = hsb.astype(_bf16)
        e16 = e_blk.astype(_bf16)
        kk = _dot(st16, wk1r[...]) + _dot(hs16, wk2r[...]) + bkr[...]
        vv = _dot(st16, wv1r[...]) + _dot(hs16, wv2r[...]) + bvr[...]
        ter[...] = _dot(e16, wecr[...]) + ber[...]
        kv0r[...] = jnp.concatenate([kk[:, :64], vv[:, :64]], axis=1)
        kv1r[...] = jnp.concatenate([kk[:, 64:], vv[:, 64:]], axis=1)
        bm = jnp.max(jnp.abs(kk), axis=0, keepdims=True)
        prev = jnp.where(i == 0, jnp.zeros_like(bm), kmr[...])
        kmr[...] = jnp.maximum(prev, bm)

    full = pl.BlockSpec((_D, _D), lambda i: (0, 0))
    row = pl.BlockSpec((1, _D), lambda i: (0, 0))
    eb = pl.BlockSpec((_RB, _D), lambda i: (i, 0))
    in_specs = [eb, eb]
    args = [hs, ein]
    if second:
        in_specs = in_specs + [eb]
        args = args + [tprev]
    in_specs = in_specs + [full, full, full, full, full, row, row, row,
                           row, row]
    args = args + [wk1, wk2, wv1, wv2, wec, bk, bv, be, g, b]
    return pl.pallas_call(
        body,
        grid=(_E // _RB,),
        in_specs=in_specs,
        out_specs=[eb, eb, eb, row],
        out_shape=[
            jax.ShapeDtypeStruct((_E, _D), _f32),
            jax.ShapeDtypeStruct((_E, _D), _f32),
            jax.ShapeDtypeStruct((_E, _D), _f32),
            jax.ShapeDtypeStruct((1, _D), _f32),
        ],
    )(*args)


def _tc_node(O0, O1, h, ww1, ww2, bw, wea, web, g, b, wqn=None, bqn=None):
    first = wqn is not None

    def body(*refs):
        if first:
            (o0r, o1r, hr, ww1r, ww2r, bwr, wear, webr, gr, br,
             wqr, bqr, hpr, pr, rr, qr, qmr) = refs
        else:
            (o0r, o1r, hr, ww1r, ww2r, bwr, wear, webr, gr, br,
             hpr, pr, rr) = refs
        i = pl.program_id(0)
        o0 = o0r[...]
        o1 = o1r[...]
        ss = jnp.concatenate([o0[:, :64], o1[:, :64]], axis=1)
        tt = jnp.concatenate([o0[:, 64:], o1[:, 64:]], axis=1)
        pos = ss > 0.0
        hn = jnp.where(pos, tt / jnp.where(pos, ss, 1.0), 0.0)
        h16 = hr[...].astype(_bf16)
        h_new = _dot(hn.astype(_bf16), ww1r[...]) + _dot(h16, ww2r[...]) \
            + bwr[...]
        hn16 = h_new.astype(_bf16)
        pr[...] = _dot(hn16, wear[...])
        rr[...] = _dot(hn16, webr[...])
        hp = _ln(jnp.maximum(h_new, 0.0), gr[...], br[...])
        hpr[...] = hp
        if first:
            q = _dot(hp.astype(_bf16), wqr[...]) + bqr[...]
            qr[...] = q
            bm = jnp.max(jnp.abs(q), axis=0, keepdims=True)
            prev = jnp.where(i == 0, jnp.zeros_like(bm), qmr[...])
            qmr[...] = jnp.maximum(prev, bm)

    full = pl.BlockSpec((_D, _D), lambda i: (0, 0))
    row = pl.BlockSpec((1, _D), lambda i: (0, 0))
    nb = pl.BlockSpec((_NB, _D), lambda i: (i, 0))
    in_specs = [nb, nb, nb, full, full, row, full, full, row, row]
    out_specs = [nb, nb, nb]
    out_shape = [jax.ShapeDtypeStruct((_N, _D), _f32)] * 3
    args = [O0, O1, h, ww1, ww2, bw, wea, web, g, b]
    if first:
        in_specs = in_specs + [full, row]
        out_specs = out_specs + [nb, row]
        out_shape = out_shape + [
            jax.ShapeDtypeStruct((_N, _D), _f32),
            jax.ShapeDtypeStruct((1, _D), _f32),
        ]
        args = args + [wqn, bqn]
    return pl.pallas_call(
        body,
        grid=(_N // _NB,),
        in_specs=in_specs,
        out_specs=out_specs,
        out_shape=out_shape,
    )(*args)


def _tc_final(esum, te, g, b):
    def body(e_ref, t_ref, gr, br, o_ref):
        o_ref[...] = _ln(jnp.maximum(e_ref[...] + t_ref[...], 0.0),
                         gr[...], br[...])

    eb = pl.BlockSpec((_RB, _D), lambda i: (i, 0))
    row = pl.BlockSpec((1, _D), lambda i: (0, 0))
    return pl.pallas_call(
        body,
        grid=(_E // _RB,),
        in_specs=[eb, eb, row, row],
        out_specs=eb,
        out_shape=jax.ShapeDtypeStruct((_E, _D), _f32),
    )(esum, te, g, b)


def kernel(x, edge_attr, edge_index, Wq0, bq0, Wk0, bk0, Wv0, bv0, Ww0,
           bw0, We0, be0, Wq1, bq1, Wk1, bk1, Wv1, bv1, Ww1, bw1, We1,
           be1, gamma, beta):
    ei = edge_index.reshape(2, _NCHUNK, 1, _CW)
    src2d, dst2d = ei[0], ei[1]
    zrows = jnp.zeros((624, _D), _f32)
    g = gamma.reshape(1, _D)
    b = beta.reshape(1, _D)

    def w16(w):
        return w.astype(_bf16)

    q, qmax = _tc_qpass(x, w16(Wq0), bq0.reshape(1, _D))
    hs = _sc_gather(x, src2d)
    kv0, kv1, te, kmax = _tc_edge(
        hs, edge_attr, w16(Wk0[:128]), w16(Wk0[128:]), w16(Wv0[:128]),
        w16(Wv0[128:]), w16(We0[256:]), bk0.reshape(1, _D),
        bv0.reshape(1, _D), be0.reshape(1, _D), g, b, second=False)
    o0, o1 = _sc_attn(kv0, kv1, q, qmax * kmax, dst2d, zrows)
    h1, p1, r1, q, qmax = _tc_node(
        o0, o1, x, w16(Ww0[:128]), w16(Ww0[128:]),
        bw0.reshape(1, _D), w16(We0[:128]), w16(We0[128:256]), g, b,
        w16(Wq1), bq1.reshape(1, _D))
    esum1, hs2 = _sc_edgec(p1, r1, h1, src2d, dst2d)

    kv0, kv1, te, kmax = _tc_edge(
        hs2, esum1, w16(Wk1[:128]), w16(Wk1[128:]), w16(Wv1[:128]),
        w16(Wv1[128:]), w16(We1[256:]), bk1.reshape(1, _D),
        bv1.reshape(1, _D), be1.reshape(1, _D), g, b, second=True,
        tprev=te)
    o0, o1 = _sc_attn(kv0, kv1, q, qmax * kmax, dst2d, zrows)
    h2, p2, r2 = _tc_node(
        o0, o1, h1, w16(Ww1[:128]), w16(Ww1[128:]),
        bw1.reshape(1, _D), w16(We1[:128]), w16(We1[128:256]), g, b)
    (esum2,) = _sc_edgec(p2, r2, None, src2d, dst2d)
    e_out = _tc_final(esum2, te, g, b)
    return h2, e_out
